# Initial kernel scaffold; baseline (speedup 1.0000x reference)
import jax
import jax.numpy as jnp
from jax.experimental import pallas as pl


def kernel(x, edge_index, bn0_g, bn0_b, W1, b1, bn1_g, bn1_b, W2, b2, bn2_g, bn2_b, cW1, cb1, cW2, cb2, bn3_g, bn3_b):
    raise NotImplementedError("write your pallas kernel here")


# speedup vs baseline: 3.8007x; 3.8007x over previous
import functools

import jax
import jax.numpy as jnp
from jax import lax
from jax.experimental import pallas as pl
from jax.experimental.pallas import tpu as pltpu
from jax.experimental.pallas import tpu_sc as plsc

N = 10000
E = 160000
H = 256
NP = 10240
EP = 163840
NSUB = 16
NCORE = 2
RPT = NP // NSUB
EPS_SUB = EP // NSUB
CH = 128
NCHUNK = EPS_SUB // CH
RB = 64

@functools.cache
def _mesh():
    return plsc.VectorSubcoreMesh(core_axis_name="c", subcore_axis_name="s")


def _fill_rows(ref, nrows, width, value):
    val = jnp.full((16,), value, jnp.float32)

    def body(i, _):
        for cb in range(width // 16):
            ref[i, pl.ds(cb * 16, 16)] = val
        return 0

    lax.fori_loop(0, nrows, body, 0)


def _deg_body(dst_hbm, deg_hbm, idx_v, ones_v, stage_v, acc):
    c = lax.axis_index("c")
    s = lax.axis_index("s")
    row0 = s * RPT

    _fill_rows(stage_v, RB, 128, 0.0)
    for b in range(RPT // RB):
        pltpu.sync_copy(stage_v, acc.at[pl.ds(row0 + b * RB, RB)])
    plsc.subcore_barrier()

    _fill_rows(ones_v, CH, 128, 1.0)
    ept = EP // (NCORE * NSUB)

    def chunk(k, _):
        e0 = (c * NSUB + s) * ept + k * CH
        pltpu.sync_copy(dst_hbm.at[pl.ds(e0, CH)], idx_v)
        pltpu.sync_copy(ones_v, acc.at[idx_v], add=True)
        return 0

    lax.fori_loop(0, ept // CH, chunk, 0)
    plsc.subcore_barrier()

    for b in range(RPT // RB):
        r = row0 + b * RB
        pltpu.sync_copy(acc.at[pl.ds(r, RB)], stage_v)
        pltpu.sync_copy(stage_v, deg_hbm.at[pl.ds(c * NP + r, RB)])


@functools.cache
def _deg_kernel():
    return pl.kernel(
        _deg_body,
        out_type=jax.ShapeDtypeStruct((NCORE * NP, 128), jnp.float32),
        mesh=_mesh(),
        scratch_types=[
            pltpu.VMEM((CH,), jnp.int32),
            pltpu.VMEM((CH, 128), jnp.float32),
            pltpu.VMEM((RB, 128), jnp.float32),
            pltpu.VMEM_SHARED((NP, 128), jnp.float32),
        ],
    )


def _scale_rows(stage_v, d_v, fh):
    def body(i, _):
        splat = d_v[i]
        for cb in range(fh // 16):
            sl = pl.ds(cb * 16, 16)
            stage_v[i, sl] = stage_v[i, sl] * splat
        return 0

    lax.fori_loop(0, RB, body, 0)


def _make_prop_body(split):
    fh = 128

    def body(y0, srcv, dst, d2_hbm, d1_hbm, p_out, t_out,
             idx_s, idx_d, rows_v, stage_v, d_v, acc, sem):
        c = lax.axis_index("c")
        s = lax.axis_index("s")
        tbase = (c * NP) if split else 0
        sbase = (c * EP) if split else 0
        row0 = s * RPT

        if split:
            wrow0 = row0
            nwb = RPT // RB
        else:
            wrow0 = c * (NP // 2) + s * (NP // 32)
            nwb = (NP // 32) // RB

        def hop(table, dest, d_hbm):
            for b in range(RPT // RB):
                r = row0 + b * RB
                pltpu.sync_copy(table.at[pl.ds(tbase + r, RB)], stage_v)
                pltpu.sync_copy(stage_v, acc.at[pl.ds(r, RB)])
            plsc.subcore_barrier()

            def chunk(k, _):
                e0 = s * EPS_SUB + k * CH
                pltpu.sync_copy(srcv.at[pl.ds(sbase + e0, CH)], idx_s)
                pltpu.async_copy(table.at[idx_s], rows_v, sem).wait()
                pltpu.sync_copy(dst.at[pl.ds(e0, CH)], idx_d)
                pltpu.sync_copy(rows_v, acc.at[idx_d], add=True)
                return 0

            lax.fori_loop(0, NCHUNK, chunk, 0)
            plsc.subcore_barrier()

            for b in range(nwb):
                r = wrow0 + b * RB
                pltpu.sync_copy(acc.at[pl.ds(r, RB)], stage_v)
                pltpu.sync_copy(d_hbm.at[pl.ds(r, RB)], d_v)
                _scale_rows(stage_v, d_v, fh)
                pltpu.sync_copy(stage_v, dest.at[pl.ds(tbase + r, RB)])
            plsc.subcore_barrier()

        hop(y0, p_out, d2_hbm)
        hop(p_out, t_out, d2_hbm)
        hop(t_out, p_out, d1_hbm)

    return body


@functools.cache
def _make_prop_kernel(split):
    fh = 128
    nrow = (NCORE * NP) if split else NP
    return pl.kernel(
        _make_prop_body(split),
        out_type=[
            jax.ShapeDtypeStruct((nrow, fh), jnp.float32),
            jax.ShapeDtypeStruct((nrow, fh), jnp.float32),
        ],
        mesh=_mesh(),
        scratch_types=[
            pltpu.VMEM((CH,), jnp.int32),
            pltpu.VMEM((CH,), jnp.int32),
            pltpu.VMEM((CH, fh), jnp.float32),
            pltpu.VMEM((RB, fh), jnp.float32),
            pltpu.VMEM((RB, 16), jnp.float32),
            pltpu.VMEM_SHARED((NP, fh), jnp.float32),
            pltpu.SemaphoreType.DMA,
        ],
    )


def _edge_gather_body(z_hbm, src_hbm, dst_hbm, va_hbm, vb_hbm,
                      idx_v, rows_v, sem):
    c = lax.axis_index("c")
    s = lax.axis_index("s")
    wid = s * NCORE + c
    ept = EP // (NCORE * NSUB)

    def chunk(k, _):
        e0 = wid * ept + k * CH
        pltpu.sync_copy(src_hbm.at[pl.ds(e0, CH)], idx_v)
        pltpu.async_copy(z_hbm.at[idx_v], rows_v, sem).wait()
        pltpu.sync_copy(rows_v, va_hbm.at[pl.ds(e0, CH)])
        pltpu.sync_copy(dst_hbm.at[pl.ds(e0, CH)], idx_v)
        pltpu.async_copy(z_hbm.at[idx_v], rows_v, sem).wait()
        pltpu.sync_copy(rows_v, vb_hbm.at[pl.ds(e0, CH)])
        return 0

    lax.fori_loop(0, ept // CH, chunk, 0)


@functools.cache
def _edge_gather():
    return pl.kernel(
        _edge_gather_body,
        out_type=[
            jax.ShapeDtypeStruct((EP, 128), jnp.float32),
            jax.ShapeDtypeStruct((EP, 128), jnp.float32),
        ],
        mesh=_mesh(),
        scratch_types=[
            pltpu.VMEM((CH,), jnp.int32),
            pltpu.VMEM((CH, 128), jnp.float32),
            pltpu.SemaphoreType.DMA,
        ],
    )


def _bn_cols(x, g, b):
    m = jnp.mean(x, axis=0, keepdims=True)
    v = jnp.mean((x - m) ** 2, axis=0, keepdims=True)
    return (x - m) * lax.rsqrt(v + 1e-5) * g + b


def _dot(a, b):
    return jnp.dot(a, b, preferred_element_type=jnp.float32,
                   precision=lax.Precision.HIGHEST)


def _bn_affine(m, v, g, b):
    sc = g * lax.rsqrt(v + 1e-5)
    return sc, b - m * sc


def _tc_a0_body(x_ref, deg_ref, g_ref, b_ref, sc_ref, sh_ref, d1_ref, d2_ref):
    deg = deg_ref[0:NP, 0:1] + deg_ref[NP:2 * NP, 0:1] + 1.0
    dinv = lax.rsqrt(deg)
    d1_ref[...] = jnp.broadcast_to(dinv, (NP, 16))
    d2_ref[...] = jnp.broadcast_to(1.0 / deg, (NP, 16))
    x = x_ref[...]
    m = jnp.mean(x, axis=0, keepdims=True)
    v = jnp.mean((x - m) ** 2, axis=0, keepdims=True)
    sc_ref[...], sh_ref[...] = _bn_affine(m, v, g_ref[...], b_ref[...])


_tc_a0 = pl.pallas_call(
    _tc_a0_body,
    out_shape=[
        jax.ShapeDtypeStruct((1, H), jnp.float32),
        jax.ShapeDtypeStruct((1, H), jnp.float32),
        jax.ShapeDtypeStruct((NP, 16), jnp.float32),
        jax.ShapeDtypeStruct((NP, 16), jnp.float32),
    ],
)

_BLK_A = 1000


def _tc_a2_body(x_ref, sc_ref, sh_ref, w_ref, d1_ref, ya_ref, yb_ref):
    h = x_ref[...] * sc_ref[...] + sh_ref[...]
    y = _dot(h, w_ref[...].T) * d1_ref[:, 0:1]
    ya_ref[...] = y[:, :128]
    yb_ref[...] = y[:, 128:]


_tc_a2 = pl.pallas_call(
    _tc_a2_body,
    grid=(N // _BLK_A,),
    in_specs=[
        pl.BlockSpec((_BLK_A, H), lambda i: (i, 0)),
        pl.BlockSpec((1, H), lambda i: (0, 0)),
        pl.BlockSpec((1, H), lambda i: (0, 0)),
        pl.BlockSpec((H, H), lambda i: (0, 0)),
        pl.BlockSpec((_BLK_A, 16), lambda i: (i, 0)),
    ],
    out_specs=[
        pl.BlockSpec((_BLK_A, 128), lambda i: (i, 0)),
        pl.BlockSpec((_BLK_A, 128), lambda i: (i, 0)),
    ],
    out_shape=[
        jax.ShapeDtypeStruct((N, 128), jnp.float32),
        jax.ShapeDtypeStruct((N, 128), jnp.float32),
    ],
)


def _tc_c0_body(p_ref, b1_ref, g_ref, bb_ref, sc_ref, sh_ref):
    h = jnp.concatenate([p_ref[0:N], p_ref[NP:NP + N]], axis=1) + b1_ref[...]
    r = jax.nn.relu(h)
    m = jnp.mean(r, axis=0, keepdims=True)
    v = jnp.mean((r - m) ** 2, axis=0, keepdims=True)
    sc_ref[...], sh_ref[...] = _bn_affine(m, v, g_ref[...], bb_ref[...])


_tc_c0 = pl.pallas_call(
    _tc_c0_body,
    out_shape=[
        jax.ShapeDtypeStruct((1, H), jnp.float32),
        jax.ShapeDtypeStruct((1, H), jnp.float32),
    ],
)

_BLK_C = 1024


def _tc_c2_body(pa_ref, pb_ref, b1_ref, sc_ref, sh_ref, w_ref, d1_ref, y_ref):
    h = jnp.concatenate([pa_ref[...], pb_ref[...]], axis=1) + b1_ref[...]
    r = jax.nn.relu(h) * sc_ref[...] + sh_ref[...]
    y_ref[...] = _dot(r, w_ref[...].T) * d1_ref[:, 0:1]


_tc_c2 = pl.pallas_call(
    _tc_c2_body,
    grid=(NP // _BLK_C,),
    in_specs=[
        pl.BlockSpec((_BLK_C, 128), lambda i: (i, 0)),
        pl.BlockSpec((_BLK_C, 128), lambda i: (i + NP // _BLK_C, 0)),
        pl.BlockSpec((1, H), lambda i: (0, 0)),
        pl.BlockSpec((1, H), lambda i: (0, 0)),
        pl.BlockSpec((1, H), lambda i: (0, 0)),
        pl.BlockSpec((128, H), lambda i: (0, 0)),
        pl.BlockSpec((_BLK_C, 16), lambda i: (i, 0)),
    ],
    out_specs=pl.BlockSpec((_BLK_C, 128), lambda i: (i, 0)),
    out_shape=jax.ShapeDtypeStruct((NP, 128), jnp.float32),
)


def _tc_e0_body(p_ref, b2_ref, g_ref, bb_ref, sc_ref, sh_ref):
    r = jax.nn.relu(p_ref[0:N] + b2_ref[...])
    m = jnp.mean(r, axis=0, keepdims=True)
    v = jnp.mean((r - m) ** 2, axis=0, keepdims=True)
    sc_ref[...], sh_ref[...] = _bn_affine(m, v, g_ref[...], bb_ref[...])


_tc_e0 = pl.pallas_call(
    _tc_e0_body,
    out_shape=[
        jax.ShapeDtypeStruct((1, 128), jnp.float32),
        jax.ShapeDtypeStruct((1, 128), jnp.float32),
    ],
)


def _tc_e2_body(p_ref, b2_ref, sc_ref, sh_ref, cw1_ref, cb1_ref,
                cw2_ref, cb2_ref, z_ref):
    r = jax.nn.relu(p_ref[...] + b2_ref[...]) * sc_ref[...] + sh_ref[...]
    t = jax.nn.relu(_dot(r, cw1_ref[...].T) + cb1_ref[...])
    z = _dot(t, cw2_ref[...].T) + cb2_ref[...]
    z_ref[...] = jnp.pad(z, ((0, 0), (0, 96)))


_tc_e2 = pl.pallas_call(
    _tc_e2_body,
    grid=(NP // _BLK_C,),
    in_specs=[
        pl.BlockSpec((_BLK_C, 128), lambda i: (i, 0)),
        pl.BlockSpec((1, 128), lambda i: (0, 0)),
        pl.BlockSpec((1, 128), lambda i: (0, 0)),
        pl.BlockSpec((1, 128), lambda i: (0, 0)),
        pl.BlockSpec((64, 128), lambda i: (0, 0)),
        pl.BlockSpec((1, 64), lambda i: (0, 0)),
        pl.BlockSpec((32, 64), lambda i: (0, 0)),
        pl.BlockSpec((1, 32), lambda i: (0, 0)),
    ],
    out_specs=pl.BlockSpec((_BLK_C, 128), lambda i: (i, 0)),
    out_shape=jax.ShapeDtypeStruct((NP, 128), jnp.float32),
)


def _tc_cos_body(va_ref, vb_ref, cos_ref):
    va = va_ref[:, 0:32]
    vb = vb_ref[:, 0:32]
    dot = jnp.sum(va * vb, axis=1, keepdims=True)
    na = jnp.sqrt(jnp.sum(va * va, axis=1, keepdims=True))
    nb = jnp.sqrt(jnp.sum(vb * vb, axis=1, keepdims=True))
    eps = 1e-8
    cos = dot / (jnp.maximum(na, eps) * jnp.maximum(nb, eps))
    cos_ref[...] = cos.reshape(_BLK_E // 256, 256)


_BLK_E = 8192
_tc_cos = pl.pallas_call(
    _tc_cos_body,
    grid=(EP // _BLK_E,),
    in_specs=[
        pl.BlockSpec((_BLK_E, 128), lambda i: (i, 0)),
        pl.BlockSpec((_BLK_E, 128), lambda i: (i, 0)),
    ],
    out_specs=pl.BlockSpec((_BLK_E // 256, 256), lambda i: (i, 0)),
    out_shape=jax.ShapeDtypeStruct((EP // 256, 256), jnp.float32),
)


def _tc_g2_body(cos_ref, g_ref, b_ref, out_ref):
    c = cos_ref[0:E // 256]
    m = jnp.mean(c)
    v = jnp.mean((c - m) ** 2)
    out_ref[...] = jax.nn.sigmoid(
        (c - m) * lax.rsqrt(v + 1e-5) * g_ref[0, 0] + b_ref[0, 0])


_tc_g2 = pl.pallas_call(
    _tc_g2_body,
    out_shape=jax.ShapeDtypeStruct((E // 256, 256), jnp.float32),
)


def kernel(x, edge_index, bn0_g, bn0_b, W1, b1, bn1_g, bn1_b, W2, b2,
           bn2_g, bn2_b, cW1, cb1, cW2, cb2, bn3_g, bn3_b):
    ei = edge_index.astype(jnp.int32)
    src = ei[0]
    dst = ei[1]
    npad = EP - E
    srcp = jnp.concatenate([src, jnp.zeros((npad,), jnp.int32)])
    dstp = jnp.concatenate([dst, jnp.full((npad,), N, jnp.int32)])
    src2 = jnp.concatenate([srcp, srcp + NP])

    deg16 = _deg_kernel()(dstp)
    sc0, sh0, d1, d2 = _tc_a0(x, deg16, bn0_g.reshape(1, H),
                              bn0_b.reshape(1, H))
    y0a, y0b = _tc_a2(x, sc0, sh0, W1, d1[:N])
    zp = jnp.zeros((NP - N, 128), jnp.float32)
    y0 = jnp.concatenate([y0a, zp, y0b, zp])

    p1, _ = _make_prop_kernel(True)(y0, src2, dstp, d2, d1)
    sc1, sh1 = _tc_c0(p1, b1.reshape(1, H), bn1_g.reshape(1, H),
                      bn1_b.reshape(1, H))
    y2 = _tc_c2(p1, p1, b1.reshape(1, H), sc1, sh1, W2, d1)
    p2, _ = _make_prop_kernel(False)(y2, srcp, dstp, d2, d1)
    sc2, sh2 = _tc_e0(p2, b2.reshape(1, 128), bn2_g.reshape(1, 128),
                      bn2_b.reshape(1, 128))
    z = _tc_e2(p2, b2.reshape(1, 128), sc2, sh2, cW1, cb1.reshape(1, 64),
               cW2, cb2.reshape(1, 32))

    va, vb = _edge_gather()(z, srcp, dstp)
    cos = _tc_cos(va, vb)
    out = _tc_g2(cos, bn3_g.reshape(1, 1), bn3_b.reshape(1, 1))
    return out.reshape(E, 1)


# speedup vs baseline: 4.3968x; 1.1568x over previous
import functools

import jax
import jax.numpy as jnp
from jax import lax
from jax.experimental import pallas as pl
from jax.experimental.pallas import tpu as pltpu
from jax.experimental.pallas import tpu_sc as plsc

N = 10000
E = 160000
H = 256
NP = 10240
EP = 163840
NSUB = 16
NCORE = 2
RPT = NP // NSUB
EPS_SUB = EP // NSUB
CH = 128
NCHUNK = EPS_SUB // CH
RB = 32

@functools.cache
def _mesh():
    return plsc.VectorSubcoreMesh(core_axis_name="c", subcore_axis_name="s")


def _fill_rows(ref, nrows, width, value):
    val = jnp.full((16,), value, jnp.float32)

    def body(i, _):
        for cb in range(width // 16):
            ref[i, pl.ds(cb * 16, 16)] = val
        return 0

    lax.fori_loop(0, nrows, body, 0)


def _deg_body(dst_hbm, deg_hbm, idx_v, ones_v, stage_v, acc):
    c = lax.axis_index("c")
    s = lax.axis_index("s")
    row0 = s * RPT

    _fill_rows(stage_v, RB, 128, 0.0)
    for b in range(RPT // RB):
        pltpu.sync_copy(stage_v, acc.at[pl.ds(row0 + b * RB, RB)])
    plsc.subcore_barrier()

    _fill_rows(ones_v, CH, 128, 1.0)
    ept = EP // (NCORE * NSUB)

    def chunk(k, _):
        e0 = (c * NSUB + s) * ept + k * CH
        pltpu.sync_copy(dst_hbm.at[pl.ds(e0, CH)], idx_v)
        pltpu.sync_copy(ones_v, acc.at[idx_v], add=True)
        return 0

    lax.fori_loop(0, ept // CH, chunk, 0)
    plsc.subcore_barrier()

    for b in range(RPT // RB):
        r = row0 + b * RB
        pltpu.sync_copy(acc.at[pl.ds(r, RB)], stage_v)
        pltpu.sync_copy(stage_v, deg_hbm.at[pl.ds(c * NP + r, RB)])


@functools.cache
def _deg_kernel():
    return pl.kernel(
        _deg_body,
        out_type=jax.ShapeDtypeStruct((NCORE * NP, 128), jnp.float32),
        mesh=_mesh(),
        scratch_types=[
            pltpu.VMEM((CH,), jnp.int32),
            pltpu.VMEM((CH, 128), jnp.float32),
            pltpu.VMEM((RB, 128), jnp.float32),
            pltpu.VMEM_SHARED((NP, 128), jnp.float32),
        ],
    )


def _scale_rows(stage_v, d_v, fh):
    def body(i, _):
        splat = d_v[i]
        for cb in range(fh // 16):
            sl = pl.ds(cb * 16, 16)
            stage_v[i, sl] = stage_v[i, sl] * splat
        return 0

    lax.fori_loop(0, RB, body, 0)


def _make_prop_body(split):
    fh = 128

    def body(y0, srcv, dst, d2_hbm, d1_hbm, p_out, t_out,
             idx_s2, idx_d2, rows0, rows1, stage_v, d_v, acc,
             sg0, sg1, sw0, sw1):
        c = lax.axis_index("c")
        s = lax.axis_index("s")
        tbase = (c * NP) if split else 0
        row0 = s * RPT
        sroff = (c * (EP // CH) if split else 0) + s * (EPS_SUB // CH)
        droff = s * (EPS_SUB // CH)
        rows = (rows0, rows1)
        sems_g = (sg0, sg1)
        sems_w = (sw0, sw1)

        if split:
            wrow0 = row0
            nwb = RPT // RB
        else:
            wrow0 = c * (NP // 2) + s * (NP // 32)
            nwb = (NP // 32) // RB

        def hop(table, dest, d_hbm):
            for b in range(RPT // RB):
                r = row0 + b * RB
                pltpu.sync_copy(table.at[pl.ds(tbase + r, RB)], stage_v)
                pltpu.sync_copy(stage_v, acc.at[pl.ds(r, RB)])
            plsc.subcore_barrier()

            def blk(kk, _):
                pltpu.sync_copy(srcv.at[pl.ds(sroff + kk * 8, 8)], idx_s2)
                pltpu.sync_copy(dst.at[pl.ds(droff + kk * 8, 8)], idx_d2)
                gd = [None, None]
                wd = [None, None]
                gd[0] = pltpu.async_copy(table.at[idx_s2.at[0]], rows[0],
                                         sems_g[0])
                for j in range(1, 9):
                    b = j % 2
                    pb = 1 - b
                    if j < 8:
                        if wd[b] is not None:
                            wd[b].wait()
                        gd[b] = pltpu.async_copy(table.at[idx_s2.at[j]],
                                                 rows[b], sems_g[b])
                    gd[pb].wait()
                    wd[pb] = pltpu.async_copy(rows[pb],
                                              acc.at[idx_d2.at[j - 1]],
                                              sems_w[pb], add=True)
                wd[0].wait()
                wd[1].wait()
                return 0

            lax.fori_loop(0, NCHUNK // 8, blk, 0)
            plsc.subcore_barrier()

            for b in range(nwb):
                r = wrow0 + b * RB
                pltpu.sync_copy(acc.at[pl.ds(r, RB)], stage_v)
                pltpu.sync_copy(d_hbm.at[pl.ds(r, RB)], d_v)
                _scale_rows(stage_v, d_v, fh)
                pltpu.sync_copy(stage_v, dest.at[pl.ds(tbase + r, RB)])
            plsc.subcore_barrier()

        hop(y0, p_out, d2_hbm)
        hop(p_out, t_out, d2_hbm)
        hop(t_out, p_out, d1_hbm)

    return body


@functools.cache
def _make_prop_kernel(split):
    fh = 128
    nrow = (NCORE * NP) if split else NP
    return pl.kernel(
        _make_prop_body(split),
        out_type=[
            jax.ShapeDtypeStruct((nrow, fh), jnp.float32),
            jax.ShapeDtypeStruct((nrow, fh), jnp.float32),
        ],
        mesh=_mesh(),
        scratch_types=[
            pltpu.VMEM((8, CH), jnp.int32),
            pltpu.VMEM((8, CH), jnp.int32),
            pltpu.VMEM((CH, fh), jnp.float32),
            pltpu.VMEM((CH, fh), jnp.float32),
            pltpu.VMEM((RB, fh), jnp.float32),
            pltpu.VMEM((RB, 16), jnp.float32),
            pltpu.VMEM_SHARED((NP, fh), jnp.float32),
            pltpu.SemaphoreType.DMA,
            pltpu.SemaphoreType.DMA,
            pltpu.SemaphoreType.DMA,
            pltpu.SemaphoreType.DMA,
        ],
    )


def _edge_gather_body(z_hbm, src_hbm, dst_hbm, va_hbm, vb_hbm,
                      idx2, rows0, rows1, sg0, sg1, sw0, sw1):
    c = lax.axis_index("c")
    s = lax.axis_index("s")
    wid = s * NCORE + c
    ept = EP // (NCORE * NSUB)
    rows = (rows0, rows1)
    sems_g = (sg0, sg1)
    sems_w = (sw0, sw1)

    def pass_(idx_hbm, out_hbm):
        def blk(kk, _):
            r0 = wid * (ept // CH) + kk * 8
            pltpu.sync_copy(idx_hbm.at[pl.ds(r0, 8)], idx2)
            gd = [None, None]
            wd = [None, None]
            gd[0] = pltpu.async_copy(z_hbm.at[idx2.at[0]], rows[0], sems_g[0])
            for j in range(1, 9):
                b = j % 2
                pb = 1 - b
                if j < 8:
                    if wd[b] is not None:
                        wd[b].wait()
                    gd[b] = pltpu.async_copy(z_hbm.at[idx2.at[j]], rows[b],
                                             sems_g[b])
                gd[pb].wait()
                e0 = (r0 + j - 1) * CH
                wd[pb] = pltpu.async_copy(rows[pb], out_hbm.at[pl.ds(e0, CH)],
                                          sems_w[pb])
            wd[0].wait()
            wd[1].wait()
            return 0

        lax.fori_loop(0, ept // CH // 8, blk, 0)

    pass_(src_hbm, va_hbm)
    pass_(dst_hbm, vb_hbm)


@functools.cache
def _edge_gather():
    return pl.kernel(
        _edge_gather_body,
        out_type=[
            jax.ShapeDtypeStruct((EP, 128), jnp.float32),
            jax.ShapeDtypeStruct((EP, 128), jnp.float32),
        ],
        mesh=_mesh(),
        scratch_types=[
            pltpu.VMEM((8, CH), jnp.int32),
            pltpu.VMEM((CH, 128), jnp.float32),
            pltpu.VMEM((CH, 128), jnp.float32),
            pltpu.SemaphoreType.DMA,
            pltpu.SemaphoreType.DMA,
            pltpu.SemaphoreType.DMA,
            pltpu.SemaphoreType.DMA,
        ],
    )


def _bn_cols(x, g, b):
    m = jnp.mean(x, axis=0, keepdims=True)
    v = jnp.mean((x - m) ** 2, axis=0, keepdims=True)
    return (x - m) * lax.rsqrt(v + 1e-5) * g + b


def _dot(a, b):
    return jnp.dot(a, b, preferred_element_type=jnp.float32,
                   precision=lax.Precision.HIGHEST)


def _bn_affine(m, v, g, b):
    sc = g * lax.rsqrt(v + 1e-5)
    return sc, b - m * sc


def _tc_a0_body(x_ref, deg_ref, g_ref, b_ref, sc_ref, sh_ref, d1_ref, d2_ref):
    deg = deg_ref[0:NP, 0:1] + deg_ref[NP:2 * NP, 0:1] + 1.0
    dinv = lax.rsqrt(deg)
    d1_ref[...] = jnp.broadcast_to(dinv, (NP, 16))
    d2_ref[...] = jnp.broadcast_to(1.0 / deg, (NP, 16))
    x = x_ref[...]
    m = jnp.mean(x, axis=0, keepdims=True)
    v = jnp.mean((x - m) ** 2, axis=0, keepdims=True)
    sc_ref[...], sh_ref[...] = _bn_affine(m, v, g_ref[...], b_ref[...])


_tc_a0 = pl.pallas_call(
    _tc_a0_body,
    out_shape=[
        jax.ShapeDtypeStruct((1, H), jnp.float32),
        jax.ShapeDtypeStruct((1, H), jnp.float32),
        jax.ShapeDtypeStruct((NP, 16), jnp.float32),
        jax.ShapeDtypeStruct((NP, 16), jnp.float32),
    ],
)

_BLK_A = 1000


def _tc_a2_body(x_ref, sc_ref, sh_ref, w_ref, d1_ref, ya_ref, yb_ref):
    h = x_ref[...] * sc_ref[...] + sh_ref[...]
    y = _dot(h, w_ref[...].T) * d1_ref[:, 0:1]
    ya_ref[...] = y[:, :128]
    yb_ref[...] = y[:, 128:]


_tc_a2 = pl.pallas_call(
    _tc_a2_body,
    grid=(N // _BLK_A,),
    in_specs=[
        pl.BlockSpec((_BLK_A, H), lambda i: (i, 0)),
        pl.BlockSpec((1, H), lambda i: (0, 0)),
        pl.BlockSpec((1, H), lambda i: (0, 0)),
        pl.BlockSpec((H, H), lambda i: (0, 0)),
        pl.BlockSpec((_BLK_A, 16), lambda i: (i, 0)),
    ],
    out_specs=[
        pl.BlockSpec((_BLK_A, 128), lambda i: (i, 0)),
        pl.BlockSpec((_BLK_A, 128), lambda i: (i, 0)),
    ],
    out_shape=[
        jax.ShapeDtypeStruct((N, 128), jnp.float32),
        jax.ShapeDtypeStruct((N, 128), jnp.float32),
    ],
)


def _tc_c0_body(p_ref, b1_ref, g_ref, bb_ref, sc_ref, sh_ref):
    h = jnp.concatenate([p_ref[0:N], p_ref[NP:NP + N]], axis=1) + b1_ref[...]
    r = jax.nn.relu(h)
    m = jnp.mean(r, axis=0, keepdims=True)
    v = jnp.mean((r - m) ** 2, axis=0, keepdims=True)
    sc_ref[...], sh_ref[...] = _bn_affine(m, v, g_ref[...], bb_ref[...])


_tc_c0 = pl.pallas_call(
    _tc_c0_body,
    out_shape=[
        jax.ShapeDtypeStruct((1, H), jnp.float32),
        jax.ShapeDtypeStruct((1, H), jnp.float32),
    ],
)

_BLK_C = 1024


def _tc_c2_body(pa_ref, pb_ref, b1_ref, sc_ref, sh_ref, w_ref, d1_ref, y_ref):
    h = jnp.concatenate([pa_ref[...], pb_ref[...]], axis=1) + b1_ref[...]
    r = jax.nn.relu(h) * sc_ref[...] + sh_ref[...]
    y_ref[...] = _dot(r, w_ref[...].T) * d1_ref[:, 0:1]


_tc_c2 = pl.pallas_call(
    _tc_c2_body,
    grid=(NP // _BLK_C,),
    in_specs=[
        pl.BlockSpec((_BLK_C, 128), lambda i: (i, 0)),
        pl.BlockSpec((_BLK_C, 128), lambda i: (i + NP // _BLK_C, 0)),
        pl.BlockSpec((1, H), lambda i: (0, 0)),
        pl.BlockSpec((1, H), lambda i: (0, 0)),
        pl.BlockSpec((1, H), lambda i: (0, 0)),
        pl.BlockSpec((128, H), lambda i: (0, 0)),
        pl.BlockSpec((_BLK_C, 16), lambda i: (i, 0)),
    ],
    out_specs=pl.BlockSpec((_BLK_C, 128), lambda i: (i, 0)),
    out_shape=jax.ShapeDtypeStruct((NP, 128), jnp.float32),
)


def _tc_e0_body(p_ref, b2_ref, g_ref, bb_ref, sc_ref, sh_ref):
    r = jax.nn.relu(p_ref[0:N] + b2_ref[...])
    m = jnp.mean(r, axis=0, keepdims=True)
    v = jnp.mean((r - m) ** 2, axis=0, keepdims=True)
    sc_ref[...], sh_ref[...] = _bn_affine(m, v, g_ref[...], bb_ref[...])


_tc_e0 = pl.pallas_call(
    _tc_e0_body,
    out_shape=[
        jax.ShapeDtypeStruct((1, 128), jnp.float32),
        jax.ShapeDtypeStruct((1, 128), jnp.float32),
    ],
)


def _tc_e2_body(p_ref, b2_ref, sc_ref, sh_ref, cw1_ref, cb1_ref,
                cw2_ref, cb2_ref, z_ref):
    r = jax.nn.relu(p_ref[...] + b2_ref[...]) * sc_ref[...] + sh_ref[...]
    t = jax.nn.relu(_dot(r, cw1_ref[...].T) + cb1_ref[...])
    z = _dot(t, cw2_ref[...].T) + cb2_ref[...]
    z_ref[...] = jnp.pad(z, ((0, 0), (0, 96)))


_tc_e2 = pl.pallas_call(
    _tc_e2_body,
    grid=(NP // _BLK_C,),
    in_specs=[
        pl.BlockSpec((_BLK_C, 128), lambda i: (i, 0)),
        pl.BlockSpec((1, 128), lambda i: (0, 0)),
        pl.BlockSpec((1, 128), lambda i: (0, 0)),
        pl.BlockSpec((1, 128), lambda i: (0, 0)),
        pl.BlockSpec((64, 128), lambda i: (0, 0)),
        pl.BlockSpec((1, 64), lambda i: (0, 0)),
        pl.BlockSpec((32, 64), lambda i: (0, 0)),
        pl.BlockSpec((1, 32), lambda i: (0, 0)),
    ],
    out_specs=pl.BlockSpec((_BLK_C, 128), lambda i: (i, 0)),
    out_shape=jax.ShapeDtypeStruct((NP, 128), jnp.float32),
)


def _tc_cos_body(va_ref, vb_ref, cos_ref):
    va = va_ref[:, 0:32]
    vb = vb_ref[:, 0:32]
    dot = jnp.sum(va * vb, axis=1, keepdims=True)
    na = jnp.sqrt(jnp.sum(va * va, axis=1, keepdims=True))
    nb = jnp.sqrt(jnp.sum(vb * vb, axis=1, keepdims=True))
    eps = 1e-8
    cos = dot / (jnp.maximum(na, eps) * jnp.maximum(nb, eps))
    cos_ref[...] = cos.reshape(_BLK_E // 256, 256)


_BLK_E = 8192
_tc_cos = pl.pallas_call(
    _tc_cos_body,
    grid=(EP // _BLK_E,),
    in_specs=[
        pl.BlockSpec((_BLK_E, 128), lambda i: (i, 0)),
        pl.BlockSpec((_BLK_E, 128), lambda i: (i, 0)),
    ],
    out_specs=pl.BlockSpec((_BLK_E // 256, 256), lambda i: (i, 0)),
    out_shape=jax.ShapeDtypeStruct((EP // 256, 256), jnp.float32),
)


def _tc_g2_body(cos_ref, g_ref, b_ref, out_ref):
    c = cos_ref[0:E // 256]
    m = jnp.mean(c)
    v = jnp.mean((c - m) ** 2)
    out_ref[...] = jax.nn.sigmoid(
        (c - m) * lax.rsqrt(v + 1e-5) * g_ref[0, 0] + b_ref[0, 0])


_tc_g2 = pl.pallas_call(
    _tc_g2_body,
    out_shape=jax.ShapeDtypeStruct((E // 256, 256), jnp.float32),
)


def kernel(x, edge_index, bn0_g, bn0_b, W1, b1, bn1_g, bn1_b, W2, b2,
           bn2_g, bn2_b, cW1, cb1, cW2, cb2, bn3_g, bn3_b):
    ei = edge_index.astype(jnp.int32)
    src = ei[0]
    dst = ei[1]
    npad = EP - E
    srcp = jnp.concatenate([src, jnp.zeros((npad,), jnp.int32)])
    dstp = jnp.concatenate([dst, jnp.full((npad,), N, jnp.int32)])
    src2 = jnp.concatenate([srcp, srcp + NP])
    src2d = srcp.reshape(EP // CH, CH)
    dst2d = dstp.reshape(EP // CH, CH)
    src22d = src2.reshape(2 * EP // CH, CH)

    deg16 = _deg_kernel()(dstp)
    sc0, sh0, d1, d2 = _tc_a0(x, deg16, bn0_g.reshape(1, H),
                              bn0_b.reshape(1, H))
    y0a, y0b = _tc_a2(x, sc0, sh0, W1, d1[:N])
    zp = jnp.zeros((NP - N, 128), jnp.float32)
    y0 = jnp.concatenate([y0a, zp, y0b, zp])

    p1, _ = _make_prop_kernel(True)(y0, src22d, dst2d, d2, d1)
    sc1, sh1 = _tc_c0(p1, b1.reshape(1, H), bn1_g.reshape(1, H),
                      bn1_b.reshape(1, H))
    y2 = _tc_c2(p1, p1, b1.reshape(1, H), sc1, sh1, W2, d1)
    p2, _ = _make_prop_kernel(False)(y2, src2d, dst2d, d2, d1)
    sc2, sh2 = _tc_e0(p2, b2.reshape(1, 128), bn2_g.reshape(1, 128),
                      bn2_b.reshape(1, 128))
    z = _tc_e2(p2, b2.reshape(1, 128), sc2, sh2, cW1, cb1.reshape(1, 64),
               cW2, cb2.reshape(1, 32))

    va, vb = _edge_gather()(z, src2d, dst2d)
    cos = _tc_cos(va, vb)
    out = _tc_g2(cos, bn3_g.reshape(1, 1), bn3_b.reshape(1, 1))
    return out.reshape(E, 1)


# speedup vs baseline: 5.2717x; 1.1990x over previous
import functools

import jax
import jax.numpy as jnp
from jax import lax
from jax.experimental import pallas as pl
from jax.experimental.pallas import tpu as pltpu
from jax.experimental.pallas import tpu_sc as plsc

N = 10000
E = 160000
H = 256
NP = 10240
EP = 163840
NSUB = 16
NCORE = 2
RPT = NP // NSUB
EPS_SUB = EP // NSUB
CH = 128
NCHUNK = EPS_SUB // CH
RB = 32

@functools.cache
def _mesh():
    return plsc.VectorSubcoreMesh(core_axis_name="c", subcore_axis_name="s")


def _fill_rows(ref, nrows, width, value):
    val = jnp.full((16,), value, jnp.float32)

    def body(i, _):
        for cb in range(width // 16):
            ref[i, pl.ds(cb * 16, 16)] = val
        return 0

    lax.fori_loop(0, nrows, body, 0)


def _deg_body(dst_hbm, deg_hbm, idx_v, ones_v, stage_v, acc):
    c = lax.axis_index("c")
    s = lax.axis_index("s")
    row0 = s * RPT

    _fill_rows(stage_v, RB, 128, 0.0)
    for b in range(RPT // RB):
        pltpu.sync_copy(stage_v, acc.at[pl.ds(row0 + b * RB, RB)])
    plsc.subcore_barrier()

    _fill_rows(ones_v, CH, 128, 1.0)
    ept = EP // (NCORE * NSUB)

    def chunk(k, _):
        e0 = (c * NSUB + s) * ept + k * CH
        pltpu.sync_copy(dst_hbm.at[pl.ds(e0, CH)], idx_v)
        pltpu.sync_copy(ones_v, acc.at[idx_v], add=True)
        return 0

    lax.fori_loop(0, ept // CH, chunk, 0)
    plsc.subcore_barrier()

    for b in range(RPT // RB):
        r = row0 + b * RB
        pltpu.sync_copy(acc.at[pl.ds(r, RB)], stage_v)
        pltpu.sync_copy(stage_v, deg_hbm.at[pl.ds(c * NP + r, RB)])


@functools.cache
def _deg_kernel():
    return pl.kernel(
        _deg_body,
        out_type=jax.ShapeDtypeStruct((NCORE * NP, 128), jnp.float32),
        mesh=_mesh(),
        scratch_types=[
            pltpu.VMEM((CH,), jnp.int32),
            pltpu.VMEM((CH, 128), jnp.float32),
            pltpu.VMEM((RB, 128), jnp.float32),
            pltpu.VMEM_SHARED((NP, 128), jnp.float32),
        ],
    )


def _scale_rows(stage_v, d_v, fh):
    def body(i, _):
        splat = d_v[i]
        for cb in range(fh // 16):
            sl = pl.ds(cb * 16, 16)
            stage_v[i, sl] = stage_v[i, sl] * splat
        return 0

    lax.fori_loop(0, RB, body, 0)


def _make_prop_body(split):
    fh = 128

    def body(y0, srcv, dst, d2_hbm, d1_hbm, p_out, t_out,
             idx_s2, idx_d2, rows0, rows1, stage_v, d_v, acc,
             sg0, sg1, sw0, sw1):
        c = lax.axis_index("c")
        s = lax.axis_index("s")
        tbase = (c * NP) if split else 0
        row0 = s * RPT
        sroff = (c * (EP // CH) if split else 0) + s * (EPS_SUB // CH)
        droff = s * (EPS_SUB // CH)
        rows = (rows0, rows1)
        sems_g = (sg0, sg1)
        sems_w = (sw0, sw1)

        if split:
            wrow0 = row0
            nwb = RPT // RB
        else:
            wrow0 = c * (NP // 2) + s * (NP // 32)
            nwb = (NP // 32) // RB

        def hop(table, dest, d_hbm):
            for b in range(RPT // RB):
                r = row0 + b * RB
                pltpu.sync_copy(table.at[pl.ds(tbase + r, RB)], stage_v)
                pltpu.sync_copy(stage_v, acc.at[pl.ds(r, RB)])
            plsc.subcore_barrier()

            def blk(kk, _):
                pltpu.sync_copy(srcv.at[pl.ds(sroff + kk * 8, 8)], idx_s2)
                pltpu.sync_copy(dst.at[pl.ds(droff + kk * 8, 8)], idx_d2)
                gd = [None, None]
                wd = [None, None]
                gd[0] = pltpu.async_copy(table.at[idx_s2.at[0]], rows[0],
                                         sems_g[0])
                for j in range(1, 9):
                    b = j % 2
                    pb = 1 - b
                    if j < 8:
                        if wd[b] is not None:
                            wd[b].wait()
                        gd[b] = pltpu.async_copy(table.at[idx_s2.at[j]],
                                                 rows[b], sems_g[b])
                    gd[pb].wait()
                    wd[pb] = pltpu.async_copy(rows[pb],
                                              acc.at[idx_d2.at[j - 1]],
                                              sems_w[pb], add=True)
                wd[0].wait()
                wd[1].wait()
                return 0

            lax.fori_loop(0, NCHUNK // 8, blk, 0)
            plsc.subcore_barrier()

            for b in range(nwb):
                r = wrow0 + b * RB
                pltpu.sync_copy(acc.at[pl.ds(r, RB)], stage_v)
                pltpu.sync_copy(d_hbm.at[pl.ds(r, RB)], d_v)
                _scale_rows(stage_v, d_v, fh)
                pltpu.sync_copy(stage_v, dest.at[pl.ds(tbase + r, RB)])
            plsc.subcore_barrier()

        hop(y0, p_out, d2_hbm)
        hop(p_out, t_out, d2_hbm)
        hop(t_out, p_out, d1_hbm)

    return body


@functools.cache
def _make_prop_kernel(split):
    fh = 128
    nrow = (NCORE * NP) if split else NP
    return pl.kernel(
        _make_prop_body(split),
        out_type=[
            jax.ShapeDtypeStruct((nrow, fh), jnp.float32),
            jax.ShapeDtypeStruct((nrow, fh), jnp.float32),
        ],
        mesh=_mesh(),
        scratch_types=[
            pltpu.VMEM((8, CH), jnp.int32),
            pltpu.VMEM((8, CH), jnp.int32),
            pltpu.VMEM((CH, fh), jnp.float32),
            pltpu.VMEM((CH, fh), jnp.float32),
            pltpu.VMEM((RB, fh), jnp.float32),
            pltpu.VMEM((RB, 16), jnp.float32),
            pltpu.VMEM_SHARED((NP, fh), jnp.float32),
            pltpu.SemaphoreType.DMA,
            pltpu.SemaphoreType.DMA,
            pltpu.SemaphoreType.DMA,
            pltpu.SemaphoreType.DMA,
        ],
    )


def _hop_half_body(table, srcv, dst, d_hbm, out,
                   idx_s2, idx_d2, rows0, rows1, stage_v, d_v, acc,
                   sg0, sg1, sw0, sw1):
    c = lax.axis_index("c")
    s = lax.axis_index("s")
    row0 = s * RPT
    rows = (rows0, rows1)
    sems_g = (sg0, sg1)
    sems_w = (sw0, sw1)

    @pl.when(c == 0)
    def _init_tab():
        for b in range(RPT // RB):
            r = row0 + b * RB
            pltpu.sync_copy(table.at[pl.ds(r, RB)], stage_v)
            pltpu.sync_copy(stage_v, acc.at[pl.ds(r, RB)])

    @pl.when(c == 1)
    def _init_zero():
        _fill_rows(stage_v, RB, 128, 0.0)
        for b in range(RPT // RB):
            pltpu.sync_copy(stage_v, acc.at[pl.ds(row0 + b * RB, RB)])

    plsc.subcore_barrier()

    nck = EP // (NCORE * NSUB) // CH

    def blk(kk, _):
        r0 = (c * NSUB + s) * nck + kk * 8
        pltpu.sync_copy(srcv.at[pl.ds(r0, 8)], idx_s2)
        pltpu.sync_copy(dst.at[pl.ds(r0, 8)], idx_d2)
        gd = [None, None]
        wd = [None, None]
        gd[0] = pltpu.async_copy(table.at[idx_s2.at[0]], rows[0], sems_g[0])
        for j in range(1, 9):
            b = j % 2
            pb = 1 - b
            if j < 8:
                if wd[b] is not None:
                    wd[b].wait()
                gd[b] = pltpu.async_copy(table.at[idx_s2.at[j]], rows[b],
                                         sems_g[b])
            gd[pb].wait()
            wd[pb] = pltpu.async_copy(rows[pb], acc.at[idx_d2.at[j - 1]],
                                      sems_w[pb], add=True)
        wd[0].wait()
        wd[1].wait()
        return 0

    lax.fori_loop(0, nck // 8, blk, 0)
    plsc.subcore_barrier()

    for b in range(RPT // RB):
        r = row0 + b * RB
        pltpu.sync_copy(acc.at[pl.ds(r, RB)], stage_v)
        pltpu.sync_copy(d_hbm.at[pl.ds(r, RB)], d_v)
        _scale_rows(stage_v, d_v, 128)
        pltpu.sync_copy(stage_v, out.at[pl.ds(c * NP + r, RB)])


@functools.cache
def _hop_half_kernel():
    return pl.kernel(
        _hop_half_body,
        out_type=jax.ShapeDtypeStruct((NCORE * NP, 128), jnp.float32),
        mesh=_mesh(),
        scratch_types=[
            pltpu.VMEM((8, CH), jnp.int32),
            pltpu.VMEM((8, CH), jnp.int32),
            pltpu.VMEM((CH, 128), jnp.float32),
            pltpu.VMEM((CH, 128), jnp.float32),
            pltpu.VMEM((RB, 128), jnp.float32),
            pltpu.VMEM((RB, 16), jnp.float32),
            pltpu.VMEM_SHARED((NP, 128), jnp.float32),
            pltpu.SemaphoreType.DMA,
            pltpu.SemaphoreType.DMA,
            pltpu.SemaphoreType.DMA,
            pltpu.SemaphoreType.DMA,
        ],
    )


_BLK_TC = 1024


def _tc_comb_body(pa_ref, pb_ref, y_ref):
    y_ref[...] = pa_ref[...] + pb_ref[...]


_tc_comb = pl.pallas_call(
    _tc_comb_body,
    grid=(NP // _BLK_TC,),
    in_specs=[
        pl.BlockSpec((_BLK_TC, 128), lambda i: (i, 0)),
        pl.BlockSpec((_BLK_TC, 128), lambda i: (i + NP // _BLK_TC, 0)),
    ],
    out_specs=pl.BlockSpec((_BLK_TC, 128), lambda i: (i, 0)),
    out_shape=jax.ShapeDtypeStruct((NP, 128), jnp.float32),
)


def _edge_gather_body(z_hbm, src_hbm, dst_hbm, va_hbm, vb_hbm,
                      idx2, rows0, rows1, sg0, sg1, sw0, sw1):
    c = lax.axis_index("c")
    s = lax.axis_index("s")
    wid = s * NCORE + c
    ept = EP // (NCORE * NSUB)
    rows = (rows0, rows1)
    sems_g = (sg0, sg1)
    sems_w = (sw0, sw1)

    def pass_(idx_hbm, out_hbm):
        def blk(kk, _):
            r0 = wid * (ept // CH) + kk * 8
            pltpu.sync_copy(idx_hbm.at[pl.ds(r0, 8)], idx2)
            gd = [None, None]
            wd = [None, None]
            gd[0] = pltpu.async_copy(z_hbm.at[idx2.at[0]], rows[0], sems_g[0])
            for j in range(1, 9):
                b = j % 2
                pb = 1 - b
                if j < 8:
                    if wd[b] is not None:
                        wd[b].wait()
                    gd[b] = pltpu.async_copy(z_hbm.at[idx2.at[j]], rows[b],
                                             sems_g[b])
                gd[pb].wait()
                e0 = (r0 + j - 1) * CH
                wd[pb] = pltpu.async_copy(rows[pb], out_hbm.at[pl.ds(e0, CH)],
                                          sems_w[pb])
            wd[0].wait()
            wd[1].wait()
            return 0

        lax.fori_loop(0, ept // CH // 8, blk, 0)

    pass_(src_hbm, va_hbm)
    pass_(dst_hbm, vb_hbm)


@functools.cache
def _edge_gather():
    return pl.kernel(
        _edge_gather_body,
        out_type=[
            jax.ShapeDtypeStruct((EP, 128), jnp.float32),
            jax.ShapeDtypeStruct((EP, 128), jnp.float32),
        ],
        mesh=_mesh(),
        scratch_types=[
            pltpu.VMEM((8, CH), jnp.int32),
            pltpu.VMEM((CH, 128), jnp.float32),
            pltpu.VMEM((CH, 128), jnp.float32),
            pltpu.SemaphoreType.DMA,
            pltpu.SemaphoreType.DMA,
            pltpu.SemaphoreType.DMA,
            pltpu.SemaphoreType.DMA,
        ],
    )


def _bn_cols(x, g, b):
    m = jnp.mean(x, axis=0, keepdims=True)
    v = jnp.mean((x - m) ** 2, axis=0, keepdims=True)
    return (x - m) * lax.rsqrt(v + 1e-5) * g + b


def _dot(a, b):
    return jnp.dot(a, b, preferred_element_type=jnp.float32,
                   precision=lax.Precision.HIGHEST)


def _bn_affine(m, v, g, b):
    sc = g * lax.rsqrt(v + 1e-5)
    return sc, b - m * sc


def _tc_a0_body(x_ref, deg_ref, g_ref, b_ref, sc_ref, sh_ref, d1_ref, d2_ref):
    deg = deg_ref[0:NP, 0:1] + deg_ref[NP:2 * NP, 0:1] + 1.0
    dinv = lax.rsqrt(deg)
    d1_ref[...] = jnp.broadcast_to(dinv, (NP, 16))
    d2_ref[...] = jnp.broadcast_to(1.0 / deg, (NP, 16))
    x = x_ref[...]
    m = jnp.mean(x, axis=0, keepdims=True)
    v = jnp.mean((x - m) ** 2, axis=0, keepdims=True)
    sc_ref[...], sh_ref[...] = _bn_affine(m, v, g_ref[...], b_ref[...])


_tc_a0 = pl.pallas_call(
    _tc_a0_body,
    out_shape=[
        jax.ShapeDtypeStruct((1, H), jnp.float32),
        jax.ShapeDtypeStruct((1, H), jnp.float32),
        jax.ShapeDtypeStruct((NP, 16), jnp.float32),
        jax.ShapeDtypeStruct((NP, 16), jnp.float32),
    ],
)

_BLK_A = 1000


def _tc_a2_body(x_ref, sc_ref, sh_ref, w_ref, d1_ref, ya_ref, yb_ref):
    h = x_ref[...] * sc_ref[...] + sh_ref[...]
    y = _dot(h, w_ref[...].T) * d1_ref[:, 0:1]
    ya_ref[...] = y[:, :128]
    yb_ref[...] = y[:, 128:]


_tc_a2 = pl.pallas_call(
    _tc_a2_body,
    grid=(N // _BLK_A,),
    in_specs=[
        pl.BlockSpec((_BLK_A, H), lambda i: (i, 0)),
        pl.BlockSpec((1, H), lambda i: (0, 0)),
        pl.BlockSpec((1, H), lambda i: (0, 0)),
        pl.BlockSpec((H, H), lambda i: (0, 0)),
        pl.BlockSpec((_BLK_A, 16), lambda i: (i, 0)),
    ],
    out_specs=[
        pl.BlockSpec((_BLK_A, 128), lambda i: (i, 0)),
        pl.BlockSpec((_BLK_A, 128), lambda i: (i, 0)),
    ],
    out_shape=[
        jax.ShapeDtypeStruct((N, 128), jnp.float32),
        jax.ShapeDtypeStruct((N, 128), jnp.float32),
    ],
)


def _tc_c0_body(p_ref, b1_ref, g_ref, bb_ref, sc_ref, sh_ref):
    h = jnp.concatenate([p_ref[0:N], p_ref[NP:NP + N]], axis=1) + b1_ref[...]
    r = jax.nn.relu(h)
    m = jnp.mean(r, axis=0, keepdims=True)
    v = jnp.mean((r - m) ** 2, axis=0, keepdims=True)
    sc_ref[...], sh_ref[...] = _bn_affine(m, v, g_ref[...], bb_ref[...])


_tc_c0 = pl.pallas_call(
    _tc_c0_body,
    out_shape=[
        jax.ShapeDtypeStruct((1, H), jnp.float32),
        jax.ShapeDtypeStruct((1, H), jnp.float32),
    ],
)

_BLK_C = 1024


def _tc_c2_body(pa_ref, pb_ref, b1_ref, sc_ref, sh_ref, w_ref, d1_ref, y_ref):
    h = jnp.concatenate([pa_ref[...], pb_ref[...]], axis=1) + b1_ref[...]
    r = jax.nn.relu(h) * sc_ref[...] + sh_ref[...]
    y_ref[...] = _dot(r, w_ref[...].T) * d1_ref[:, 0:1]


_tc_c2 = pl.pallas_call(
    _tc_c2_body,
    grid=(NP // _BLK_C,),
    in_specs=[
        pl.BlockSpec((_BLK_C, 128), lambda i: (i, 0)),
        pl.BlockSpec((_BLK_C, 128), lambda i: (i + NP // _BLK_C, 0)),
        pl.BlockSpec((1, H), lambda i: (0, 0)),
        pl.BlockSpec((1, H), lambda i: (0, 0)),
        pl.BlockSpec((1, H), lambda i: (0, 0)),
        pl.BlockSpec((128, H), lambda i: (0, 0)),
        pl.BlockSpec((_BLK_C, 16), lambda i: (i, 0)),
    ],
    out_specs=pl.BlockSpec((_BLK_C, 128), lambda i: (i, 0)),
    out_shape=jax.ShapeDtypeStruct((NP, 128), jnp.float32),
)


def _tc_e0_body(p_ref, b2_ref, g_ref, bb_ref, sc_ref, sh_ref):
    r = jax.nn.relu(p_ref[0:N] + b2_ref[...])
    m = jnp.mean(r, axis=0, keepdims=True)
    v = jnp.mean((r - m) ** 2, axis=0, keepdims=True)
    sc_ref[...], sh_ref[...] = _bn_affine(m, v, g_ref[...], bb_ref[...])


_tc_e0 = pl.pallas_call(
    _tc_e0_body,
    out_shape=[
        jax.ShapeDtypeStruct((1, 128), jnp.float32),
        jax.ShapeDtypeStruct((1, 128), jnp.float32),
    ],
)


def _tc_e2_body(p_ref, b2_ref, sc_ref, sh_ref, cw1_ref, cb1_ref,
                cw2_ref, cb2_ref, z_ref):
    r = jax.nn.relu(p_ref[...] + b2_ref[...]) * sc_ref[...] + sh_ref[...]
    t = jax.nn.relu(_dot(r, cw1_ref[...].T) + cb1_ref[...])
    z = _dot(t, cw2_ref[...].T) + cb2_ref[...]
    z_ref[...] = jnp.pad(z, ((0, 0), (0, 96)))


_tc_e2 = pl.pallas_call(
    _tc_e2_body,
    grid=(NP // _BLK_C,),
    in_specs=[
        pl.BlockSpec((_BLK_C, 128), lambda i: (i, 0)),
        pl.BlockSpec((1, 128), lambda i: (0, 0)),
        pl.BlockSpec((1, 128), lambda i: (0, 0)),
        pl.BlockSpec((1, 128), lambda i: (0, 0)),
        pl.BlockSpec((64, 128), lambda i: (0, 0)),
        pl.BlockSpec((1, 64), lambda i: (0, 0)),
        pl.BlockSpec((32, 64), lambda i: (0, 0)),
        pl.BlockSpec((1, 32), lambda i: (0, 0)),
    ],
    out_specs=pl.BlockSpec((_BLK_C, 128), lambda i: (i, 0)),
    out_shape=jax.ShapeDtypeStruct((NP, 128), jnp.float32),
)


def _tc_cos_body(va_ref, vb_ref, cos_ref):
    va = va_ref[:, 0:32]
    vb = vb_ref[:, 0:32]
    dot = jnp.sum(va * vb, axis=1, keepdims=True)
    na = jnp.sqrt(jnp.sum(va * va, axis=1, keepdims=True))
    nb = jnp.sqrt(jnp.sum(vb * vb, axis=1, keepdims=True))
    eps = 1e-8
    cos = dot / (jnp.maximum(na, eps) * jnp.maximum(nb, eps))
    cos_ref[...] = cos.reshape(_BLK_E // 256, 256)


_BLK_E = 8192
_tc_cos = pl.pallas_call(
    _tc_cos_body,
    grid=(EP // _BLK_E,),
    in_specs=[
        pl.BlockSpec((_BLK_E, 128), lambda i: (i, 0)),
        pl.BlockSpec((_BLK_E, 128), lambda i: (i, 0)),
    ],
    out_specs=pl.BlockSpec((_BLK_E // 256, 256), lambda i: (i, 0)),
    out_shape=jax.ShapeDtypeStruct((EP // 256, 256), jnp.float32),
)


def _tc_g2_body(cos_ref, g_ref, b_ref, out_ref):
    c = cos_ref[0:E // 256]
    m = jnp.mean(c)
    v = jnp.mean((c - m) ** 2)
    out_ref[...] = jax.nn.sigmoid(
        (c - m) * lax.rsqrt(v + 1e-5) * g_ref[0, 0] + b_ref[0, 0])


_tc_g2 = pl.pallas_call(
    _tc_g2_body,
    out_shape=jax.ShapeDtypeStruct((E // 256, 256), jnp.float32),
)


def kernel(x, edge_index, bn0_g, bn0_b, W1, b1, bn1_g, bn1_b, W2, b2,
           bn2_g, bn2_b, cW1, cb1, cW2, cb2, bn3_g, bn3_b):
    ei = edge_index.astype(jnp.int32)
    src = ei[0]
    dst = ei[1]
    npad = EP - E
    srcp = jnp.concatenate([src, jnp.zeros((npad,), jnp.int32)])
    dstp = jnp.concatenate([dst, jnp.full((npad,), N, jnp.int32)])
    src2 = jnp.concatenate([srcp, srcp + NP])
    src2d = srcp.reshape(EP // CH, CH)
    dst2d = dstp.reshape(EP // CH, CH)
    src22d = src2.reshape(2 * EP // CH, CH)

    deg16 = _deg_kernel()(dstp)
    sc0, sh0, d1, d2 = _tc_a0(x, deg16, bn0_g.reshape(1, H),
                              bn0_b.reshape(1, H))
    y0a, y0b = _tc_a2(x, sc0, sh0, W1, d1[:N])
    zp = jnp.zeros((NP - N, 128), jnp.float32)
    y0 = jnp.concatenate([y0a, zp, y0b, zp])

    p1, _ = _make_prop_kernel(True)(y0, src22d, dst2d, d2, d1)
    sc1, sh1 = _tc_c0(p1, b1.reshape(1, H), bn1_g.reshape(1, H),
                      bn1_b.reshape(1, H))
    y2 = _tc_c2(p1, p1, b1.reshape(1, H), sc1, sh1, W2, d1)
    p2 = y2
    for dscale in (d2, d2, d1):
        partial = _hop_half_kernel()(p2, src2d, dst2d, dscale)
        p2 = _tc_comb(partial, partial)
    sc2, sh2 = _tc_e0(p2, b2.reshape(1, 128), bn2_g.reshape(1, 128),
                      bn2_b.reshape(1, 128))
    z = _tc_e2(p2, b2.reshape(1, 128), sc2, sh2, cW1, cb1.reshape(1, 64),
               cW2, cb2.reshape(1, 32))

    va, vb = _edge_gather()(z, src2d, dst2d)
    cos = _tc_cos(va, vb)
    out = _tc_g2(cos, bn3_g.reshape(1, 1), bn3_b.reshape(1, 1))
    return out.reshape(E, 1)


# speedup vs baseline: 5.4499x; 1.0338x over previous
import functools

import jax
import jax.numpy as jnp
from jax import lax
from jax.experimental import pallas as pl
from jax.experimental.pallas import tpu as pltpu
from jax.experimental.pallas import tpu_sc as plsc

N = 10000
E = 160000
H = 256
NP = 10240
EP = 163840
NSUB = 16
NCORE = 2
RPT = NP // NSUB
EPS_SUB = EP // NSUB
CH = 128
NCHUNK = EPS_SUB // CH
RB = 32

@functools.cache
def _mesh():
    return plsc.VectorSubcoreMesh(core_axis_name="c", subcore_axis_name="s")


def _fill_rows(ref, nrows, width, value):
    val = jnp.full((16,), value, jnp.float32)

    def body(i, _):
        for cb in range(width // 16):
            ref[i, pl.ds(cb * 16, 16)] = val
        return 0

    lax.fori_loop(0, nrows, body, 0)


def _deg_body(dst_hbm, deg_hbm, idx_v, ones_v, stage_v, acc):
    c = lax.axis_index("c")
    s = lax.axis_index("s")
    row0 = s * RPT

    _fill_rows(stage_v, RB, 128, 0.0)
    for b in range(RPT // RB):
        pltpu.sync_copy(stage_v, acc.at[pl.ds(row0 + b * RB, RB)])
    plsc.subcore_barrier()

    _fill_rows(ones_v, CH, 128, 1.0)
    ept = EP // (NCORE * NSUB)

    def chunk(k, _):
        e0 = (c * NSUB + s) * ept + k * CH
        pltpu.sync_copy(dst_hbm.at[pl.ds(e0, CH)], idx_v)
        pltpu.sync_copy(ones_v, acc.at[idx_v], add=True)
        return 0

    lax.fori_loop(0, ept // CH, chunk, 0)
    plsc.subcore_barrier()

    for b in range(RPT // RB):
        r = row0 + b * RB
        pltpu.sync_copy(acc.at[pl.ds(r, RB)], stage_v)
        pltpu.sync_copy(stage_v, deg_hbm.at[pl.ds(c * NP + r, RB)])


@functools.cache
def _deg_kernel():
    return pl.kernel(
        _deg_body,
        out_type=jax.ShapeDtypeStruct((NCORE * NP, 128), jnp.float32),
        mesh=_mesh(),
        scratch_types=[
            pltpu.VMEM((CH,), jnp.int32),
            pltpu.VMEM((CH, 128), jnp.float32),
            pltpu.VMEM((RB, 128), jnp.float32),
            pltpu.VMEM_SHARED((NP, 128), jnp.float32),
        ],
    )


def _scale_rows(stage_v, d_v, nrows, fh):
    def body(i, _):
        splat = d_v[i // 8, pl.ds((i % 8) * 16, 16)]
        for cb in range(fh // 16):
            sl = pl.ds(cb * 16, 16)
            stage_v[i, sl] = stage_v[i, sl] * splat
        return 0

    lax.fori_loop(0, nrows, body, 0)


RBW = 128


def _init_acc(table, acc, tbase, row0, rows, sems_g, sems_w):
    rd = [None, None]
    wr = [None, None]
    rr = [0, 0]
    nb = RPT // CH
    for b in range(nb + 1):
        if b < nb:
            bb = b % 2
            if wr[bb] is not None:
                wr[bb].wait()
            r = row0 + b * CH
            rr[bb] = r
            rd[bb] = pltpu.async_copy(table.at[pl.ds(tbase + r, CH)],
                                      rows[bb], sems_g[bb])
        if b >= 1:
            pb = (b - 1) % 2
            rd[pb].wait()
            wr[pb] = pltpu.async_copy(rows[pb], acc.at[pl.ds(rr[pb], CH)],
                                      sems_w[pb])
    for w in wr:
        if w is not None:
            w.wait()


def _scale_wb(acc, d_hbm, dest, tdest, wrow0, nblocks, rows, dbufs,
              sems_g, sems_d, sems_w, fh):
    rd = [None, None]
    dd = [None, None]
    wr = [None, None]
    rr = [0, 0]
    for b in range(nblocks + 1):
        if b < nblocks:
            bb = b % 2
            if wr[bb] is not None:
                wr[bb].wait()
            r = wrow0 + b * RBW
            rr[bb] = r
            rd[bb] = pltpu.async_copy(acc.at[pl.ds(r, RBW)],
                                      rows[bb], sems_g[bb])
            dd[bb] = pltpu.async_copy(
                d_hbm.at[pl.ds(pl.multiple_of(r // 8, RBW // 8), RBW // 8)],
                dbufs[bb], sems_d[bb])
        if b >= 1:
            pb = (b - 1) % 2
            rd[pb].wait()
            dd[pb].wait()
            _scale_rows(rows[pb], dbufs[pb], RBW, fh)
            wr[pb] = pltpu.async_copy(rows[pb],
                                      dest.at[pl.ds(tdest + rr[pb], RBW)],
                                      sems_w[pb])
    for w in wr:
        if w is not None:
            w.wait()


def _make_prop_body(split):
    fh = 128

    def body(y0, srcv, dst, d2_hbm, d1_hbm, p_out, t_out,
             idx_s2, idx_d2, rows0, rows1, dbuf0, dbuf1, acc,
             sg0, sg1, sw0, sw1, sd0, sd1):
        c = lax.axis_index("c")
        s = lax.axis_index("s")
        tbase = (c * NP) if split else 0
        row0 = s * RPT
        sroff = (c * (EP // CH) if split else 0) + s * (EPS_SUB // CH)
        droff = s * (EPS_SUB // CH)
        rows = (rows0, rows1)
        dbufs = (dbuf0, dbuf1)
        sems_g = (sg0, sg1)
        sems_w = (sw0, sw1)
        sems_d = (sd0, sd1)

        wrow0 = row0
        nwb = RPT // RBW

        def hop(table, dest, d_hbm):
            _init_acc(table, acc, tbase, row0, rows, sems_g, sems_w)
            plsc.subcore_barrier()

            def blk(kk, _):
                pltpu.sync_copy(srcv.at[pl.ds(sroff + kk * 8, 8)], idx_s2)
                pltpu.sync_copy(dst.at[pl.ds(droff + kk * 8, 8)], idx_d2)
                gd = [None, None]
                wd = [None, None]
                gd[0] = pltpu.async_copy(table.at[idx_s2.at[0]], rows[0],
                                         sems_g[0])
                for j in range(1, 9):
                    b = j % 2
                    pb = 1 - b
                    if j < 8:
                        if wd[b] is not None:
                            wd[b].wait()
                        gd[b] = pltpu.async_copy(table.at[idx_s2.at[j]],
                                                 rows[b], sems_g[b])
                    gd[pb].wait()
                    wd[pb] = pltpu.async_copy(rows[pb],
                                              acc.at[idx_d2.at[j - 1]],
                                              sems_w[pb], add=True)
                wd[0].wait()
                wd[1].wait()
                return 0

            lax.fori_loop(0, NCHUNK // 8, blk, 0)
            plsc.subcore_barrier()

            _scale_wb(acc, d_hbm, dest, tbase, wrow0, nwb, rows, dbufs,
                      sems_g, sems_d, sems_w, fh)
            plsc.subcore_barrier()

        hop(y0, p_out, d2_hbm)
        hop(p_out, t_out, d2_hbm)
        hop(t_out, p_out, d1_hbm)

    return body


@functools.cache
def _make_prop_kernel(split):
    fh = 128
    nrow = (NCORE * NP) if split else NP
    return pl.kernel(
        _make_prop_body(split),
        out_type=[
            jax.ShapeDtypeStruct((nrow, fh), jnp.float32),
            jax.ShapeDtypeStruct((nrow, fh), jnp.float32),
        ],
        mesh=_mesh(),
        scratch_types=[
            pltpu.VMEM((8, CH), jnp.int32),
            pltpu.VMEM((8, CH), jnp.int32),
            pltpu.VMEM((CH, fh), jnp.float32),
            pltpu.VMEM((CH, fh), jnp.float32),
            pltpu.VMEM((RBW // 8, 128), jnp.float32),
            pltpu.VMEM((RBW // 8, 128), jnp.float32),
            pltpu.VMEM_SHARED((NP, fh), jnp.float32),
            pltpu.SemaphoreType.DMA,
            pltpu.SemaphoreType.DMA,
            pltpu.SemaphoreType.DMA,
            pltpu.SemaphoreType.DMA,
            pltpu.SemaphoreType.DMA,
            pltpu.SemaphoreType.DMA,
        ],
    )


def _hop_half_body(table, srcv, dst, d_hbm, out,
                   idx_s2, idx_d2, rows0, rows1, dbuf0, dbuf1, acc,
                   sg0, sg1, sw0, sw1, sd0, sd1):
    c = lax.axis_index("c")
    s = lax.axis_index("s")
    row0 = s * RPT
    rows = (rows0, rows1)
    dbufs = (dbuf0, dbuf1)
    sems_g = (sg0, sg1)
    sems_w = (sw0, sw1)
    sems_d = (sd0, sd1)

    @pl.when(c == 0)
    def _init_tab():
        _init_acc(table, acc, 0, row0, rows, sems_g, sems_w)

    @pl.when(c == 1)
    def _init_zero():
        _fill_rows(rows0, CH, 128, 0.0)
        for b in range(RPT // CH):
            pltpu.sync_copy(rows0, acc.at[pl.ds(row0 + b * CH, CH)])

    plsc.subcore_barrier()

    nck = EP // (NCORE * NSUB) // CH

    def blk(kk, _):
        r0 = (c * NSUB + s) * nck + kk * 8
        pltpu.sync_copy(srcv.at[pl.ds(r0, 8)], idx_s2)
        pltpu.sync_copy(dst.at[pl.ds(r0, 8)], idx_d2)
        gd = [None, None]
        wd = [None, None]
        gd[0] = pltpu.async_copy(table.at[idx_s2.at[0]], rows[0], sems_g[0])
        for j in range(1, 9):
            b = j % 2
            pb = 1 - b
            if j < 8:
                if wd[b] is not None:
                    wd[b].wait()
                gd[b] = pltpu.async_copy(table.at[idx_s2.at[j]], rows[b],
                                         sems_g[b])
            gd[pb].wait()
            wd[pb] = pltpu.async_copy(rows[pb], acc.at[idx_d2.at[j - 1]],
                                      sems_w[pb], add=True)
        wd[0].wait()
        wd[1].wait()
        return 0

    lax.fori_loop(0, nck // 8, blk, 0)
    plsc.subcore_barrier()

    _scale_wb(acc, d_hbm, out, c * NP, row0, RPT // RBW, rows, dbufs,
              sems_g, sems_d, sems_w, 128)


@functools.cache
def _hop_half_kernel():
    return pl.kernel(
        _hop_half_body,
        out_type=jax.ShapeDtypeStruct((NCORE * NP, 128), jnp.float32),
        mesh=_mesh(),
        scratch_types=[
            pltpu.VMEM((8, CH), jnp.int32),
            pltpu.VMEM((8, CH), jnp.int32),
            pltpu.VMEM((CH, 128), jnp.float32),
            pltpu.VMEM((CH, 128), jnp.float32),
            pltpu.VMEM((RBW // 8, 128), jnp.float32),
            pltpu.VMEM((RBW // 8, 128), jnp.float32),
            pltpu.VMEM_SHARED((NP, 128), jnp.float32),
            pltpu.SemaphoreType.DMA,
            pltpu.SemaphoreType.DMA,
            pltpu.SemaphoreType.DMA,
            pltpu.SemaphoreType.DMA,
            pltpu.SemaphoreType.DMA,
            pltpu.SemaphoreType.DMA,
        ],
    )


_BLK_TC = 1024


def _tc_comb_body(pa_ref, pb_ref, y_ref):
    y_ref[...] = pa_ref[...] + pb_ref[...]


_tc_comb = pl.pallas_call(
    _tc_comb_body,
    grid=(NP // _BLK_TC,),
    in_specs=[
        pl.BlockSpec((_BLK_TC, 128), lambda i: (i, 0)),
        pl.BlockSpec((_BLK_TC, 128), lambda i: (i + NP // _BLK_TC, 0)),
    ],
    out_specs=pl.BlockSpec((_BLK_TC, 128), lambda i: (i, 0)),
    out_shape=jax.ShapeDtypeStruct((NP, 128), jnp.float32),
)


def _extract32(src_ref, dst_ref):
    def body(i, _):
        dst_ref[i, pl.ds(0, 16)] = src_ref[i, pl.ds(0, 16)]
        dst_ref[i, pl.ds(16, 16)] = src_ref[i, pl.ds(16, 16)]
        return 0

    lax.fori_loop(0, CH, body, 0)


def _edge_gather_body(z_hbm, src_hbm, dst_hbm, va_hbm, vb_hbm,
                      idx2, rows0, rows1, cmp0, cmp1, sg0, sg1, sw0, sw1):
    c = lax.axis_index("c")
    s = lax.axis_index("s")
    wid = s * NCORE + c
    ept = EP // (NCORE * NSUB)
    rows = (rows0, rows1)
    cmps = (cmp0, cmp1)
    sems_g = (sg0, sg1)
    sems_w = (sw0, sw1)

    def pass_(idx_hbm, out_hbm):
        def blk(kk, _):
            r0 = wid * (ept // CH) + kk * 8
            pltpu.sync_copy(idx_hbm.at[pl.ds(r0, 8)], idx2)
            gd = [None, None]
            wd = [None, None]
            gd[0] = pltpu.async_copy(z_hbm.at[idx2.at[0]], rows[0], sems_g[0])
            for j in range(1, 9):
                b = j % 2
                pb = 1 - b
                if j < 8:
                    gd[b] = pltpu.async_copy(z_hbm.at[idx2.at[j]], rows[b],
                                             sems_g[b])
                gd[pb].wait()
                if wd[pb] is not None:
                    wd[pb].wait()
                _extract32(rows[pb], cmps[pb])
                e0 = (r0 + j - 1) * CH
                wd[pb] = pltpu.async_copy(cmps[pb], out_hbm.at[pl.ds(e0, CH)],
                                          sems_w[pb])
            wd[0].wait()
            wd[1].wait()
            return 0

        lax.fori_loop(0, ept // CH // 8, blk, 0)

    pass_(src_hbm, va_hbm)
    pass_(dst_hbm, vb_hbm)


@functools.cache
def _edge_gather():
    return pl.kernel(
        _edge_gather_body,
        out_type=[
            jax.ShapeDtypeStruct((EP, 32), jnp.float32),
            jax.ShapeDtypeStruct((EP, 32), jnp.float32),
        ],
        mesh=_mesh(),
        scratch_types=[
            pltpu.VMEM((8, CH), jnp.int32),
            pltpu.VMEM((CH, 128), jnp.float32),
            pltpu.VMEM((CH, 128), jnp.float32),
            pltpu.VMEM((CH, 32), jnp.float32),
            pltpu.VMEM((CH, 32), jnp.float32),
            pltpu.SemaphoreType.DMA,
            pltpu.SemaphoreType.DMA,
            pltpu.SemaphoreType.DMA,
            pltpu.SemaphoreType.DMA,
        ],
    )


def _bn_cols(x, g, b):
    m = jnp.mean(x, axis=0, keepdims=True)
    v = jnp.mean((x - m) ** 2, axis=0, keepdims=True)
    return (x - m) * lax.rsqrt(v + 1e-5) * g + b


def _dot(a, b):
    return jnp.dot(a, b, preferred_element_type=jnp.float32,
                   precision=lax.Precision.HIGHEST)


def _bn_affine(m, v, g, b):
    sc = g * lax.rsqrt(v + 1e-5)
    return sc, b - m * sc


def _tc_a0_body(x_ref, deg_ref, g_ref, b_ref, sc_ref, sh_ref, d1_ref, d2_ref):
    deg = deg_ref[0:NP, 0:1] + deg_ref[NP:2 * NP, 0:1] + 1.0
    dinv = lax.rsqrt(deg)
    d1_ref[...] = jnp.broadcast_to(dinv, (NP, 16))
    d2_ref[...] = jnp.broadcast_to(1.0 / deg, (NP, 16))
    x = x_ref[...]
    m = jnp.mean(x, axis=0, keepdims=True)
    v = jnp.mean((x - m) ** 2, axis=0, keepdims=True)
    sc_ref[...], sh_ref[...] = _bn_affine(m, v, g_ref[...], b_ref[...])


_tc_a0 = pl.pallas_call(
    _tc_a0_body,
    out_shape=[
        jax.ShapeDtypeStruct((1, H), jnp.float32),
        jax.ShapeDtypeStruct((1, H), jnp.float32),
        jax.ShapeDtypeStruct((NP, 16), jnp.float32),
        jax.ShapeDtypeStruct((NP, 16), jnp.float32),
    ],
)

_BLK_A = 1000


def _tc_a2_body(x_ref, sc_ref, sh_ref, w_ref, d1_ref, ya_ref, yb_ref):
    h = x_ref[...] * sc_ref[...] + sh_ref[...]
    y = _dot(h, w_ref[...].T) * d1_ref[:, 0:1]
    ya_ref[...] = y[:, :128]
    yb_ref[...] = y[:, 128:]


_tc_a2 = pl.pallas_call(
    _tc_a2_body,
    grid=(N // _BLK_A,),
    in_specs=[
        pl.BlockSpec((_BLK_A, H), lambda i: (i, 0)),
        pl.BlockSpec((1, H), lambda i: (0, 0)),
        pl.BlockSpec((1, H), lambda i: (0, 0)),
        pl.BlockSpec((H, H), lambda i: (0, 0)),
        pl.BlockSpec((_BLK_A, 16), lambda i: (i, 0)),
    ],
    out_specs=[
        pl.BlockSpec((_BLK_A, 128), lambda i: (i, 0)),
        pl.BlockSpec((_BLK_A, 128), lambda i: (i, 0)),
    ],
    out_shape=[
        jax.ShapeDtypeStruct((N, 128), jnp.float32),
        jax.ShapeDtypeStruct((N, 128), jnp.float32),
    ],
)


def _tc_c0_body(p_ref, b1_ref, g_ref, bb_ref, sc_ref, sh_ref):
    h = jnp.concatenate([p_ref[0:N], p_ref[NP:NP + N]], axis=1) + b1_ref[...]
    r = jax.nn.relu(h)
    m = jnp.mean(r, axis=0, keepdims=True)
    v = jnp.mean((r - m) ** 2, axis=0, keepdims=True)
    sc_ref[...], sh_ref[...] = _bn_affine(m, v, g_ref[...], bb_ref[...])


_tc_c0 = pl.pallas_call(
    _tc_c0_body,
    out_shape=[
        jax.ShapeDtypeStruct((1, H), jnp.float32),
        jax.ShapeDtypeStruct((1, H), jnp.float32),
    ],
)

_BLK_C = 1024


def _tc_c2_body(pa_ref, pb_ref, b1_ref, sc_ref, sh_ref, w_ref, d1_ref, y_ref):
    h = jnp.concatenate([pa_ref[...], pb_ref[...]], axis=1) + b1_ref[...]
    r = jax.nn.relu(h) * sc_ref[...] + sh_ref[...]
    y_ref[...] = _dot(r, w_ref[...].T) * d1_ref[:, 0:1]


_tc_c2 = pl.pallas_call(
    _tc_c2_body,
    grid=(NP // _BLK_C,),
    in_specs=[
        pl.BlockSpec((_BLK_C, 128), lambda i: (i, 0)),
        pl.BlockSpec((_BLK_C, 128), lambda i: (i + NP // _BLK_C, 0)),
        pl.BlockSpec((1, H), lambda i: (0, 0)),
        pl.BlockSpec((1, H), lambda i: (0, 0)),
        pl.BlockSpec((1, H), lambda i: (0, 0)),
        pl.BlockSpec((128, H), lambda i: (0, 0)),
        pl.BlockSpec((_BLK_C, 16), lambda i: (i, 0)),
    ],
    out_specs=pl.BlockSpec((_BLK_C, 128), lambda i: (i, 0)),
    out_shape=jax.ShapeDtypeStruct((NP, 128), jnp.float32),
)


def _tc_e0_body(p_ref, b2_ref, g_ref, bb_ref, sc_ref, sh_ref):
    r = jax.nn.relu(p_ref[0:N] + b2_ref[...])
    m = jnp.mean(r, axis=0, keepdims=True)
    v = jnp.mean((r - m) ** 2, axis=0, keepdims=True)
    sc_ref[...], sh_ref[...] = _bn_affine(m, v, g_ref[...], bb_ref[...])


_tc_e0 = pl.pallas_call(
    _tc_e0_body,
    out_shape=[
        jax.ShapeDtypeStruct((1, 128), jnp.float32),
        jax.ShapeDtypeStruct((1, 128), jnp.float32),
    ],
)


def _tc_e2_body(p_ref, b2_ref, sc_ref, sh_ref, cw1_ref, cb1_ref,
                cw2_ref, cb2_ref, z_ref):
    r = jax.nn.relu(p_ref[...] + b2_ref[...]) * sc_ref[...] + sh_ref[...]
    t = jax.nn.relu(_dot(r, cw1_ref[...].T) + cb1_ref[...])
    z = _dot(t, cw2_ref[...].T) + cb2_ref[...]
    z_ref[...] = jnp.pad(z, ((0, 0), (0, 96)))


_tc_e2 = pl.pallas_call(
    _tc_e2_body,
    grid=(NP // _BLK_C,),
    in_specs=[
        pl.BlockSpec((_BLK_C, 128), lambda i: (i, 0)),
        pl.BlockSpec((1, 128), lambda i: (0, 0)),
        pl.BlockSpec((1, 128), lambda i: (0, 0)),
        pl.BlockSpec((1, 128), lambda i: (0, 0)),
        pl.BlockSpec((64, 128), lambda i: (0, 0)),
        pl.BlockSpec((1, 64), lambda i: (0, 0)),
        pl.BlockSpec((32, 64), lambda i: (0, 0)),
        pl.BlockSpec((1, 32), lambda i: (0, 0)),
    ],
    out_specs=pl.BlockSpec((_BLK_C, 128), lambda i: (i, 0)),
    out_shape=jax.ShapeDtypeStruct((NP, 128), jnp.float32),
)


def _tc_cos_body(va_ref, vb_ref, cos_ref):
    va = va_ref[...]
    vb = vb_ref[...]
    dot = jnp.sum(va * vb, axis=1, keepdims=True)
    na = jnp.sqrt(jnp.sum(va * va, axis=1, keepdims=True))
    nb = jnp.sqrt(jnp.sum(vb * vb, axis=1, keepdims=True))
    eps = 1e-8
    cos = dot / (jnp.maximum(na, eps) * jnp.maximum(nb, eps))
    cos_ref[...] = cos.reshape(_BLK_E // 256, 256)


_BLK_E = 8192
_tc_cos = pl.pallas_call(
    _tc_cos_body,
    grid=(EP // _BLK_E,),
    in_specs=[
        pl.BlockSpec((_BLK_E, 32), lambda i: (i, 0)),
        pl.BlockSpec((_BLK_E, 32), lambda i: (i, 0)),
    ],
    out_specs=pl.BlockSpec((_BLK_E // 256, 256), lambda i: (i, 0)),
    out_shape=jax.ShapeDtypeStruct((EP // 256, 256), jnp.float32),
)


def _tc_g2_body(cos_ref, g_ref, b_ref, out_ref):
    c = cos_ref[0:E // 256]
    m = jnp.mean(c)
    v = jnp.mean((c - m) ** 2)
    out_ref[...] = jax.nn.sigmoid(
        (c - m) * lax.rsqrt(v + 1e-5) * g_ref[0, 0] + b_ref[0, 0])


_tc_g2 = pl.pallas_call(
    _tc_g2_body,
    out_shape=jax.ShapeDtypeStruct((E // 256, 256), jnp.float32),
)


def kernel(x, edge_index, bn0_g, bn0_b, W1, b1, bn1_g, bn1_b, W2, b2,
           bn2_g, bn2_b, cW1, cb1, cW2, cb2, bn3_g, bn3_b):
    ei = edge_index.astype(jnp.int32)
    src = ei[0]
    dst = ei[1]
    npad = EP - E
    srcp = jnp.concatenate([src, jnp.zeros((npad,), jnp.int32)])
    dstp = jnp.concatenate([dst, jnp.full((npad,), N, jnp.int32)])
    src2 = jnp.concatenate([srcp, srcp + NP])
    src2d = srcp.reshape(EP // CH, CH)
    dst2d = dstp.reshape(EP // CH, CH)
    src22d = src2.reshape(2 * EP // CH, CH)

    deg16 = _deg_kernel()(dstp)
    sc0, sh0, d1, d2 = _tc_a0(x, deg16, bn0_g.reshape(1, H),
                              bn0_b.reshape(1, H))
    y0a, y0b = _tc_a2(x, sc0, sh0, W1, d1[:N])
    zp = jnp.zeros((NP - N, 128), jnp.float32)
    y0 = jnp.concatenate([y0a, zp, y0b, zp])

    d1p = d1.reshape(NP * 16 // 128, 128)
    d2p = d2.reshape(NP * 16 // 128, 128)
    p1, _ = _make_prop_kernel(True)(y0, src22d, dst2d, d2p, d1p)
    sc1, sh1 = _tc_c0(p1, b1.reshape(1, H), bn1_g.reshape(1, H),
                      bn1_b.reshape(1, H))
    y2 = _tc_c2(p1, p1, b1.reshape(1, H), sc1, sh1, W2, d1)
    p2 = y2
    for dscale in (d2p, d2p, d1p):
        partial = _hop_half_kernel()(p2, src2d, dst2d, dscale)
        p2 = _tc_comb(partial, partial)
    sc2, sh2 = _tc_e0(p2, b2.reshape(1, 128), bn2_g.reshape(1, 128),
                      bn2_b.reshape(1, 128))
    z = _tc_e2(p2, b2.reshape(1, 128), sc2, sh2, cW1, cb1.reshape(1, 64),
               cW2, cb2.reshape(1, 32))

    va, vb = _edge_gather()(z, src2d, dst2d)
    cos = _tc_cos(va, vb)
    out = _tc_g2(cos, bn3_g.reshape(1, 1), bn3_b.reshape(1, 1))
    return out.reshape(E, 1)


# speedup vs baseline: 6.0939x; 1.1182x over previous
import functools

import jax
import jax.numpy as jnp
from jax import lax
from jax.experimental import pallas as pl
from jax.experimental.pallas import tpu as pltpu
from jax.experimental.pallas import tpu_sc as plsc

N = 10000
E = 160000
H = 256
NP = 10240
EP = 163840
NSUB = 16
NCORE = 2
RPT = NP // NSUB
EPS_SUB = EP // NSUB
CH = 128
NCHUNK = EPS_SUB // CH
RB = 32

@functools.cache
def _mesh():
    return plsc.VectorSubcoreMesh(core_axis_name="c", subcore_axis_name="s")


def _fill_rows(ref, nrows, width, value):
    val = jnp.full((16,), value, jnp.float32)

    def body(i, _):
        for cb in range(width // 16):
            ref[i, pl.ds(cb * 16, 16)] = val
        return 0

    lax.fori_loop(0, nrows, body, 0)


def _deg_body(dst_hbm, deg_hbm, idx_v, ones_v, stage_v, acc):
    c = lax.axis_index("c")
    s = lax.axis_index("s")
    row0 = s * RPT

    _fill_rows(stage_v, RB, 128, 0.0)
    for b in range(RPT // RB):
        pltpu.sync_copy(stage_v, acc.at[pl.ds(row0 + b * RB, RB)])
    plsc.subcore_barrier()

    _fill_rows(ones_v, CH, 128, 1.0)
    ept = EP // (NCORE * NSUB)

    def chunk(k, _):
        e0 = (c * NSUB + s) * ept + k * CH
        pltpu.sync_copy(dst_hbm.at[pl.ds(e0, CH)], idx_v)
        pltpu.sync_copy(ones_v, acc.at[idx_v], add=True)
        return 0

    lax.fori_loop(0, ept // CH, chunk, 0)
    plsc.subcore_barrier()

    for b in range(RPT // RB):
        r = row0 + b * RB
        pltpu.sync_copy(acc.at[pl.ds(r, RB)], stage_v)
        pltpu.sync_copy(stage_v, deg_hbm.at[pl.ds(c * NP + r, RB)])


@functools.cache
def _deg_kernel():
    return pl.kernel(
        _deg_body,
        out_type=jax.ShapeDtypeStruct((NCORE * NP, 128), jnp.float32),
        mesh=_mesh(),
        scratch_types=[
            pltpu.VMEM((CH,), jnp.int32),
            pltpu.VMEM((CH, 128), jnp.float32),
            pltpu.VMEM((RB, 128), jnp.float32),
            pltpu.VMEM_SHARED((NP, 128), jnp.float32),
        ],
    )


def _scale_rows(stage_v, d_v, nrows, fh):
    def body(i, _):
        splat = d_v[i // 8, pl.ds((i % 8) * 16, 16)]
        for cb in range(fh // 16):
            sl = pl.ds(cb * 16, 16)
            stage_v[i, sl] = stage_v[i, sl] * splat
        return 0

    lax.fori_loop(0, nrows, body, 0)


RBW = 128


def _init_acc(table, acc, tbase, row0, rows, sems_g, sems_w):
    rd = [None, None]
    wr = [None, None]
    rr = [0, 0]
    nb = RPT // CH
    for b in range(nb + 1):
        if b < nb:
            bb = b % 2
            if wr[bb] is not None:
                wr[bb].wait()
            r = row0 + b * CH
            rr[bb] = r
            rd[bb] = pltpu.async_copy(table.at[pl.ds(tbase + r, CH)],
                                      rows[bb], sems_g[bb])
        if b >= 1:
            pb = (b - 1) % 2
            rd[pb].wait()
            wr[pb] = pltpu.async_copy(rows[pb], acc.at[pl.ds(rr[pb], CH)],
                                      sems_w[pb])
    for w in wr:
        if w is not None:
            w.wait()


def _scale_wb(acc, d_hbm, dest, tdest, wrow0, nblocks, rows, dbufs,
              sems_g, sems_d, sems_w, fh):
    rd = [None, None]
    dd = [None, None]
    wr = [None, None]
    rr = [0, 0]
    for b in range(nblocks + 1):
        if b < nblocks:
            bb = b % 2
            if wr[bb] is not None:
                wr[bb].wait()
            r = wrow0 + b * RBW
            rr[bb] = r
            rd[bb] = pltpu.async_copy(acc.at[pl.ds(r, RBW)],
                                      rows[bb], sems_g[bb])
            dd[bb] = pltpu.async_copy(
                d_hbm.at[pl.ds(pl.multiple_of(r // 8, RBW // 8), RBW // 8)],
                dbufs[bb], sems_d[bb])
        if b >= 1:
            pb = (b - 1) % 2
            rd[pb].wait()
            dd[pb].wait()
            _scale_rows(rows[pb], dbufs[pb], RBW, fh)
            wr[pb] = pltpu.async_copy(rows[pb],
                                      dest.at[pl.ds(tdest + rr[pb], RBW)],
                                      sems_w[pb])
    for w in wr:
        if w is not None:
            w.wait()


def _make_prop_body(split):
    fh = 128

    def body(y0, srcv, dst, d2_hbm, d1_hbm, p_out, t_out,
             idx_s2, idx_d2, rows0, rows1, dbuf0, dbuf1, acc,
             sg0, sg1, sw0, sw1, sd0, sd1):
        c = lax.axis_index("c")
        s = lax.axis_index("s")
        tbase = (c * NP) if split else 0
        row0 = s * RPT
        sroff = (c * (EP // CH) if split else 0) + s * (EPS_SUB // CH)
        droff = s * (EPS_SUB // CH)
        rows = (rows0, rows1)
        dbufs = (dbuf0, dbuf1)
        sems_g = (sg0, sg1)
        sems_w = (sw0, sw1)
        sems_d = (sd0, sd1)

        wrow0 = row0
        nwb = RPT // RBW

        def hop(table, dest, d_hbm):
            _init_acc(table, acc, tbase, row0, rows, sems_g, sems_w)
            plsc.subcore_barrier()

            def blk(kk, _):
                pltpu.sync_copy(srcv.at[pl.ds(sroff + kk * 8, 8)], idx_s2)
                pltpu.sync_copy(dst.at[pl.ds(droff + kk * 8, 8)], idx_d2)
                gd = [None, None]
                wd = [None, None]
                gd[0] = pltpu.async_copy(table.at[idx_s2.at[0]], rows[0],
                                         sems_g[0])
                for j in range(1, 9):
                    b = j % 2
                    pb = 1 - b
                    if j < 8:
                        if wd[b] is not None:
                            wd[b].wait()
                        gd[b] = pltpu.async_copy(table.at[idx_s2.at[j]],
                                                 rows[b], sems_g[b])
                    gd[pb].wait()
                    wd[pb] = pltpu.async_copy(rows[pb],
                                              acc.at[idx_d2.at[j - 1]],
                                              sems_w[pb], add=True)
                wd[0].wait()
                wd[1].wait()
                return 0

            lax.fori_loop(0, NCHUNK // 8, blk, 0)
            plsc.subcore_barrier()

            _scale_wb(acc, d_hbm, dest, tbase, wrow0, nwb, rows, dbufs,
                      sems_g, sems_d, sems_w, fh)
            plsc.subcore_barrier()

        hop(y0, p_out, d2_hbm)
        hop(p_out, t_out, d2_hbm)
        hop(t_out, p_out, d1_hbm)

    return body


@functools.cache
def _make_prop_kernel(split):
    fh = 128
    nrow = (NCORE * NP) if split else NP
    return pl.kernel(
        _make_prop_body(split),
        out_type=[
            jax.ShapeDtypeStruct((nrow, fh), jnp.float32),
            jax.ShapeDtypeStruct((nrow, fh), jnp.float32),
        ],
        mesh=_mesh(),
        scratch_types=[
            pltpu.VMEM((8, CH), jnp.int32),
            pltpu.VMEM((8, CH), jnp.int32),
            pltpu.VMEM((CH, fh), jnp.float32),
            pltpu.VMEM((CH, fh), jnp.float32),
            pltpu.VMEM((RBW // 8, 128), jnp.float32),
            pltpu.VMEM((RBW // 8, 128), jnp.float32),
            pltpu.VMEM_SHARED((NP, fh), jnp.float32),
            pltpu.SemaphoreType.DMA,
            pltpu.SemaphoreType.DMA,
            pltpu.SemaphoreType.DMA,
            pltpu.SemaphoreType.DMA,
            pltpu.SemaphoreType.DMA,
            pltpu.SemaphoreType.DMA,
        ],
    )


def _hop_half_body(table, srcv, dst, d_hbm, out,
                   idx_s2, idx_d2, rows0, rows1, dbuf0, dbuf1, acc,
                   sg0, sg1, sw0, sw1, sd0, sd1):
    c = lax.axis_index("c")
    s = lax.axis_index("s")
    row0 = s * RPT
    rows = (rows0, rows1)
    dbufs = (dbuf0, dbuf1)
    sems_g = (sg0, sg1)
    sems_w = (sw0, sw1)
    sems_d = (sd0, sd1)

    @pl.when(c == 0)
    def _init_tab():
        _init_acc(table, acc, 0, row0, rows, sems_g, sems_w)

    @pl.when(c == 1)
    def _init_zero():
        _fill_rows(rows0, CH, 128, 0.0)
        for b in range(RPT // CH):
            pltpu.sync_copy(rows0, acc.at[pl.ds(row0 + b * CH, CH)])

    plsc.subcore_barrier()

    nck = EP // (NCORE * NSUB) // CH

    def blk(kk, _):
        r0 = (c * NSUB + s) * nck + kk * 8
        pltpu.sync_copy(srcv.at[pl.ds(r0, 8)], idx_s2)
        pltpu.sync_copy(dst.at[pl.ds(r0, 8)], idx_d2)
        gd = [None, None]
        wd = [None, None]
        gd[0] = pltpu.async_copy(table.at[idx_s2.at[0]], rows[0], sems_g[0])
        for j in range(1, 9):
            b = j % 2
            pb = 1 - b
            if j < 8:
                if wd[b] is not None:
                    wd[b].wait()
                gd[b] = pltpu.async_copy(table.at[idx_s2.at[j]], rows[b],
                                         sems_g[b])
            gd[pb].wait()
            wd[pb] = pltpu.async_copy(rows[pb], acc.at[idx_d2.at[j - 1]],
                                      sems_w[pb], add=True)
        wd[0].wait()
        wd[1].wait()
        return 0

    lax.fori_loop(0, nck // 8, blk, 0)
    plsc.subcore_barrier()

    _scale_wb(acc, d_hbm, out, c * NP, row0, RPT // RBW, rows, dbufs,
              sems_g, sems_d, sems_w, 128)


@functools.cache
def _hop_half_kernel():
    return pl.kernel(
        _hop_half_body,
        out_type=jax.ShapeDtypeStruct((NCORE * NP, 128), jnp.float32),
        mesh=_mesh(),
        scratch_types=[
            pltpu.VMEM((8, CH), jnp.int32),
            pltpu.VMEM((8, CH), jnp.int32),
            pltpu.VMEM((CH, 128), jnp.float32),
            pltpu.VMEM((CH, 128), jnp.float32),
            pltpu.VMEM((RBW // 8, 128), jnp.float32),
            pltpu.VMEM((RBW // 8, 128), jnp.float32),
            pltpu.VMEM_SHARED((NP, 128), jnp.float32),
            pltpu.SemaphoreType.DMA,
            pltpu.SemaphoreType.DMA,
            pltpu.SemaphoreType.DMA,
            pltpu.SemaphoreType.DMA,
            pltpu.SemaphoreType.DMA,
            pltpu.SemaphoreType.DMA,
        ],
    )


_BLK_TC = 1024


def _tc_comb_body(pa_ref, pb_ref, y_ref):
    y_ref[...] = pa_ref[...] + pb_ref[...]


_tc_comb = pl.pallas_call(
    _tc_comb_body,
    grid=(NP // _BLK_TC,),
    in_specs=[
        pl.BlockSpec((_BLK_TC, 128), lambda i: (i, 0)),
        pl.BlockSpec((_BLK_TC, 128), lambda i: (i + NP // _BLK_TC, 0)),
    ],
    out_specs=pl.BlockSpec((_BLK_TC, 128), lambda i: (i, 0)),
    out_shape=jax.ShapeDtypeStruct((NP, 128), jnp.float32),
)


def _extract32(src_ref, dst_ref):
    def body(i, _):
        o = (i % 4) * 32
        dst_ref[i // 4, pl.ds(o, 16)] = src_ref[i, pl.ds(0, 16)]
        dst_ref[i // 4, pl.ds(o + 16, 16)] = src_ref[i, pl.ds(16, 16)]
        return 0

    lax.fori_loop(0, CH, body, 0)


def _edge_gather_body(z_hbm, src_hbm, dst_hbm, va_hbm, vb_hbm,
                      idx2, rows0, rows1, cmp0, cmp1, zs,
                      sg0, sg1, sw0, sw1):
    c = lax.axis_index("c")
    s = lax.axis_index("s")
    wid = s * NCORE + c
    ept = EP // (NCORE * NSUB)
    rows = (rows0, rows1)
    cmps = (cmp0, cmp1)
    sems_g = (sg0, sg1)
    sems_w = (sw0, sw1)

    _init_acc(z_hbm, zs, 0, s * RPT, rows, sems_g, sems_w)
    plsc.subcore_barrier()

    def pass_(idx_hbm, out_hbm):
        def blk(kk, _):
            r0 = wid * (ept // CH) + kk * 8
            pltpu.sync_copy(idx_hbm.at[pl.ds(r0, 8)], idx2)
            gd = [None, None]
            wd = [None, None]
            gd[0] = pltpu.async_copy(zs.at[idx2.at[0]], rows[0], sems_g[0])
            for j in range(1, 9):
                b = j % 2
                pb = 1 - b
                if j < 8:
                    gd[b] = pltpu.async_copy(zs.at[idx2.at[j]], rows[b],
                                             sems_g[b])
                gd[pb].wait()
                if wd[pb] is not None:
                    wd[pb].wait()
                _extract32(rows[pb], cmps[pb])
                e4 = (r0 + j - 1) * (CH // 4)
                wd[pb] = pltpu.async_copy(
                    cmps[pb], out_hbm.at[pl.ds(e4, CH // 4)], sems_w[pb])
            wd[0].wait()
            wd[1].wait()
            return 0

        lax.fori_loop(0, ept // CH // 8, blk, 0)

    pass_(src_hbm, va_hbm)
    pass_(dst_hbm, vb_hbm)


@functools.cache
def _edge_gather():
    return pl.kernel(
        _edge_gather_body,
        out_type=[
            jax.ShapeDtypeStruct((EP // 4, 128), jnp.float32),
            jax.ShapeDtypeStruct((EP // 4, 128), jnp.float32),
        ],
        mesh=_mesh(),
        scratch_types=[
            pltpu.VMEM((8, CH), jnp.int32),
            pltpu.VMEM((CH, 128), jnp.float32),
            pltpu.VMEM((CH, 128), jnp.float32),
            pltpu.VMEM((CH // 4, 128), jnp.float32),
            pltpu.VMEM((CH // 4, 128), jnp.float32),
            pltpu.VMEM_SHARED((NP, 128), jnp.float32),
            pltpu.SemaphoreType.DMA,
            pltpu.SemaphoreType.DMA,
            pltpu.SemaphoreType.DMA,
            pltpu.SemaphoreType.DMA,
        ],
    )


def _bn_cols(x, g, b):
    m = jnp.mean(x, axis=0, keepdims=True)
    v = jnp.mean((x - m) ** 2, axis=0, keepdims=True)
    return (x - m) * lax.rsqrt(v + 1e-5) * g + b


def _dot(a, b):
    return jnp.dot(a, b, preferred_element_type=jnp.float32,
                   precision=lax.Precision.HIGHEST)


def _bn_affine(m, v, g, b):
    sc = g * lax.rsqrt(v + 1e-5)
    return sc, b - m * sc


def _tc_a0_body(x_ref, deg_ref, g_ref, b_ref, sc_ref, sh_ref, d1_ref, d2_ref):
    deg = deg_ref[0:NP, 0:1] + deg_ref[NP:2 * NP, 0:1] + 1.0
    dinv = lax.rsqrt(deg)
    d1_ref[...] = jnp.broadcast_to(dinv, (NP, 16))
    d2_ref[...] = jnp.broadcast_to(1.0 / deg, (NP, 16))
    x = x_ref[...]
    m = jnp.mean(x, axis=0, keepdims=True)
    v = jnp.mean((x - m) ** 2, axis=0, keepdims=True)
    sc_ref[...], sh_ref[...] = _bn_affine(m, v, g_ref[...], b_ref[...])


_tc_a0 = pl.pallas_call(
    _tc_a0_body,
    out_shape=[
        jax.ShapeDtypeStruct((1, H), jnp.float32),
        jax.ShapeDtypeStruct((1, H), jnp.float32),
        jax.ShapeDtypeStruct((NP, 16), jnp.float32),
        jax.ShapeDtypeStruct((NP, 16), jnp.float32),
    ],
)

_BLK_A = 1000


def _tc_a2_body(x_ref, sc_ref, sh_ref, w_ref, d1_ref, ya_ref, yb_ref):
    h = x_ref[...] * sc_ref[...] + sh_ref[...]
    y = _dot(h, w_ref[...].T) * d1_ref[:, 0:1]
    ya_ref[...] = y[:, :128]
    yb_ref[...] = y[:, 128:]


_tc_a2 = pl.pallas_call(
    _tc_a2_body,
    grid=(N // _BLK_A,),
    in_specs=[
        pl.BlockSpec((_BLK_A, H), lambda i: (i, 0)),
        pl.BlockSpec((1, H), lambda i: (0, 0)),
        pl.BlockSpec((1, H), lambda i: (0, 0)),
        pl.BlockSpec((H, H), lambda i: (0, 0)),
        pl.BlockSpec((_BLK_A, 16), lambda i: (i, 0)),
    ],
    out_specs=[
        pl.BlockSpec((_BLK_A, 128), lambda i: (i, 0)),
        pl.BlockSpec((_BLK_A, 128), lambda i: (i, 0)),
    ],
    out_shape=[
        jax.ShapeDtypeStruct((N, 128), jnp.float32),
        jax.ShapeDtypeStruct((N, 128), jnp.float32),
    ],
)


def _tc_c0_body(p_ref, b1_ref, g_ref, bb_ref, sc_ref, sh_ref):
    h = jnp.concatenate([p_ref[0:N], p_ref[NP:NP + N]], axis=1) + b1_ref[...]
    r = jax.nn.relu(h)
    m = jnp.mean(r, axis=0, keepdims=True)
    v = jnp.mean((r - m) ** 2, axis=0, keepdims=True)
    sc_ref[...], sh_ref[...] = _bn_affine(m, v, g_ref[...], bb_ref[...])


_tc_c0 = pl.pallas_call(
    _tc_c0_body,
    out_shape=[
        jax.ShapeDtypeStruct((1, H), jnp.float32),
        jax.ShapeDtypeStruct((1, H), jnp.float32),
    ],
)

_BLK_C = 1024


def _tc_c2_body(pa_ref, pb_ref, b1_ref, sc_ref, sh_ref, w_ref, d1_ref, y_ref):
    h = jnp.concatenate([pa_ref[...], pb_ref[...]], axis=1) + b1_ref[...]
    r = jax.nn.relu(h) * sc_ref[...] + sh_ref[...]
    y_ref[...] = _dot(r, w_ref[...].T) * d1_ref[:, 0:1]


_tc_c2 = pl.pallas_call(
    _tc_c2_body,
    grid=(NP // _BLK_C,),
    in_specs=[
        pl.BlockSpec((_BLK_C, 128), lambda i: (i, 0)),
        pl.BlockSpec((_BLK_C, 128), lambda i: (i + NP // _BLK_C, 0)),
        pl.BlockSpec((1, H), lambda i: (0, 0)),
        pl.BlockSpec((1, H), lambda i: (0, 0)),
        pl.BlockSpec((1, H), lambda i: (0, 0)),
        pl.BlockSpec((128, H), lambda i: (0, 0)),
        pl.BlockSpec((_BLK_C, 16), lambda i: (i, 0)),
    ],
    out_specs=pl.BlockSpec((_BLK_C, 128), lambda i: (i, 0)),
    out_shape=jax.ShapeDtypeStruct((NP, 128), jnp.float32),
)


def _tc_e0_body(p_ref, b2_ref, g_ref, bb_ref, sc_ref, sh_ref):
    r = jax.nn.relu(p_ref[0:N] + b2_ref[...])
    m = jnp.mean(r, axis=0, keepdims=True)
    v = jnp.mean((r - m) ** 2, axis=0, keepdims=True)
    sc_ref[...], sh_ref[...] = _bn_affine(m, v, g_ref[...], bb_ref[...])


_tc_e0 = pl.pallas_call(
    _tc_e0_body,
    out_shape=[
        jax.ShapeDtypeStruct((1, 128), jnp.float32),
        jax.ShapeDtypeStruct((1, 128), jnp.float32),
    ],
)


def _tc_e2_body(p_ref, b2_ref, sc_ref, sh_ref, cw1_ref, cb1_ref,
                cw2_ref, cb2_ref, z_ref):
    r = jax.nn.relu(p_ref[...] + b2_ref[...]) * sc_ref[...] + sh_ref[...]
    t = jax.nn.relu(_dot(r, cw1_ref[...].T) + cb1_ref[...])
    z = _dot(t, cw2_ref[...].T) + cb2_ref[...]
    z_ref[...] = jnp.pad(z, ((0, 0), (0, 96)))


_tc_e2 = pl.pallas_call(
    _tc_e2_body,
    grid=(NP // _BLK_C,),
    in_specs=[
        pl.BlockSpec((_BLK_C, 128), lambda i: (i, 0)),
        pl.BlockSpec((1, 128), lambda i: (0, 0)),
        pl.BlockSpec((1, 128), lambda i: (0, 0)),
        pl.BlockSpec((1, 128), lambda i: (0, 0)),
        pl.BlockSpec((64, 128), lambda i: (0, 0)),
        pl.BlockSpec((1, 64), lambda i: (0, 0)),
        pl.BlockSpec((32, 64), lambda i: (0, 0)),
        pl.BlockSpec((1, 32), lambda i: (0, 0)),
    ],
    out_specs=pl.BlockSpec((_BLK_C, 128), lambda i: (i, 0)),
    out_shape=jax.ShapeDtypeStruct((NP, 128), jnp.float32),
)


def _tc_cos_body(va_ref, vb_ref, cos_ref):
    va = va_ref[...]
    vb = vb_ref[...]
    dot = jnp.sum(va * vb, axis=1, keepdims=True)
    na = jnp.sqrt(jnp.sum(va * va, axis=1, keepdims=True))
    nb = jnp.sqrt(jnp.sum(vb * vb, axis=1, keepdims=True))
    eps = 1e-8
    cos = dot / (jnp.maximum(na, eps) * jnp.maximum(nb, eps))
    cos_ref[...] = cos.reshape(_BLK_E // 256, 256)


_BLK_E = 8192
_tc_cos = pl.pallas_call(
    _tc_cos_body,
    grid=(EP // _BLK_E,),
    in_specs=[
        pl.BlockSpec((_BLK_E, 32), lambda i: (i, 0)),
        pl.BlockSpec((_BLK_E, 32), lambda i: (i, 0)),
    ],
    out_specs=pl.BlockSpec((_BLK_E // 256, 256), lambda i: (i, 0)),
    out_shape=jax.ShapeDtypeStruct((EP // 256, 256), jnp.float32),
)


def _tc_g2_body(cos_ref, g_ref, b_ref, out_ref):
    c = cos_ref[0:E // 256]
    m = jnp.mean(c)
    v = jnp.mean((c - m) ** 2)
    out_ref[...] = jax.nn.sigmoid(
        (c - m) * lax.rsqrt(v + 1e-5) * g_ref[0, 0] + b_ref[0, 0])


_tc_g2 = pl.pallas_call(
    _tc_g2_body,
    out_shape=jax.ShapeDtypeStruct((E // 256, 256), jnp.float32),
)


def kernel(x, edge_index, bn0_g, bn0_b, W1, b1, bn1_g, bn1_b, W2, b2,
           bn2_g, bn2_b, cW1, cb1, cW2, cb2, bn3_g, bn3_b):
    ei = edge_index.astype(jnp.int32)
    src = ei[0]
    dst = ei[1]
    npad = EP - E
    srcp = jnp.concatenate([src, jnp.zeros((npad,), jnp.int32)])
    dstp = jnp.concatenate([dst, jnp.full((npad,), N, jnp.int32)])
    src2 = jnp.concatenate([srcp, srcp + NP])
    src2d = srcp.reshape(EP // CH, CH)
    dst2d = dstp.reshape(EP // CH, CH)
    src22d = src2.reshape(2 * EP // CH, CH)

    deg16 = _deg_kernel()(dstp)
    sc0, sh0, d1, d2 = _tc_a0(x, deg16, bn0_g.reshape(1, H),
                              bn0_b.reshape(1, H))
    y0a, y0b = _tc_a2(x, sc0, sh0, W1, d1[:N])
    zp = jnp.zeros((NP - N, 128), jnp.float32)
    y0 = jnp.concatenate([y0a, zp, y0b, zp])

    d1p = d1.reshape(NP * 16 // 128, 128)
    d2p = d2.reshape(NP * 16 // 128, 128)
    p1, _ = _make_prop_kernel(True)(y0, src22d, dst2d, d2p, d1p)
    sc1, sh1 = _tc_c0(p1, b1.reshape(1, H), bn1_g.reshape(1, H),
                      bn1_b.reshape(1, H))
    y2 = _tc_c2(p1, p1, b1.reshape(1, H), sc1, sh1, W2, d1)
    p2 = y2
    for dscale in (d2p, d2p, d1p):
        partial = _hop_half_kernel()(p2, src2d, dst2d, dscale)
        p2 = _tc_comb(partial, partial)
    sc2, sh2 = _tc_e0(p2, b2.reshape(1, 128), bn2_g.reshape(1, 128),
                      bn2_b.reshape(1, 128))
    z = _tc_e2(p2, b2.reshape(1, 128), sc2, sh2, cW1, cb1.reshape(1, 64),
               cW2, cb2.reshape(1, 32))

    va, vb = _edge_gather()(z, src2d, dst2d)
    cos = _tc_cos(va.reshape(EP, 32), vb.reshape(EP, 32))
    out = _tc_g2(cos, bn3_g.reshape(1, 1), bn3_b.reshape(1, 1))
    return out.reshape(E, 1)
